# Initial kernel scaffold; baseline (speedup 1.0000x reference)
import jax
import jax.numpy as jnp
from jax.experimental import pallas as pl


def kernel(x1, adj1_idx, adj1_val, x2, adj2_idx, adj2_val, w_e1_1, w_e1_2, w_e1_3, w_d1_1, w_d1_2, w_d1_3, w_e2_1, w_e2_2, w_e2_3, w_d2_1, w_d2_2, w_d2_3, centers1, centers2):
    raise NotImplementedError("write your pallas kernel here")


# speedup vs baseline: 3.1205x; 3.1205x over previous
import functools

import jax
import jax.numpy as jnp
from jax import lax
from jax.experimental import pallas as pl
from jax.experimental.pallas import tpu as pltpu
from jax.experimental.pallas import tpu_sc as plsc

_N = 4096
_E = 65536
_NZ = 32
_NCL = 10

_NC = 2
_NS = 16
_LANES = 16
_NW = _NC * _NS
_EPW = _E // _NW
_CB = 128
_TPC = _EPW // _CB
_RPS = _N // _NS
_ZR = 64


@functools.lru_cache(maxsize=None)
def _make_spmm(width):
    mesh = plsc.VectorSubcoreMesh(core_axis_name="c", subcore_axis_name="s")
    jw = width // _LANES

    def body(y_hbm, idx_hbm, val_hbm, out_hbm,
             acc_sh, rows_v, col_v, row_v, val_v, zbuf, sem):
        cid = lax.axis_index("c")
        sid = lax.axis_index("s")
        wid = sid * _NC + cid

        zero16 = jnp.zeros((_LANES,), jnp.float32)

        def zrow(r, carry):
            for j in range(jw):
                zbuf[r, pl.ds(j * _LANES, _LANES)] = zero16
            return carry

        lax.fori_loop(0, _ZR, zrow, 0)
        for rr in range(_RPS // _ZR):
            pltpu.sync_copy(zbuf, acc_sh.at[pl.ds(sid * _RPS + rr * _ZR, _ZR)])
        plsc.subcore_barrier()

        def chunk(t, carry):
            base = wid * _EPW + t * _CB
            pltpu.sync_copy(idx_hbm.at[0, pl.ds(base, _CB)], row_v)
            pltpu.sync_copy(idx_hbm.at[1, pl.ds(base, _CB)], col_v)
            pltpu.sync_copy(val_hbm.at[pl.ds(base, _CB)], val_v)
            pltpu.async_copy(y_hbm.at[col_v], rows_v, sem).wait()

            def scale(g, c2):
                vv = val_v[pl.ds(g * _LANES, _LANES)]
                for k in range(_LANES):
                    v = vv[k]
                    e = g * _LANES + k
                    for j in range(jw):
                        sl = pl.ds(j * _LANES, _LANES)
                        rows_v[e, sl] = rows_v[e, sl] * v
                return c2

            lax.fori_loop(0, _CB // _LANES, scale, 0)
            pltpu.sync_copy(rows_v, acc_sh.at[row_v], add=True)
            return carry

        lax.fori_loop(0, _TPC, chunk, 0)

        plsc.subcore_barrier()
        pltpu.sync_copy(acc_sh.at[pl.ds(sid * _RPS, _RPS)],
                        out_hbm.at[cid, pl.ds(sid * _RPS, _RPS)])

    return pl.kernel(
        body,
        out_type=jax.ShapeDtypeStruct((_NC, _N, width), jnp.float32),
        mesh=mesh,
        compiler_params=pltpu.CompilerParams(use_tc_tiling_on_sc=False),
        scratch_types=[
            pltpu.VMEM_SHARED((_N, width), jnp.float32),
            pltpu.VMEM((_CB, width), jnp.float32),
            pltpu.VMEM((_CB,), jnp.int32),
            pltpu.VMEM((_CB,), jnp.int32),
            pltpu.VMEM((_CB,), jnp.float32),
            pltpu.VMEM((_ZR, width), jnp.float32),
            pltpu.SemaphoreType.DMA,
        ],
    )


def _spmm(y, idx, val):
    return _make_spmm(y.shape[1])(y, idx, val)


_BN = 1024


def _dot_t(a, b):
    return lax.dot_general(a, b, (((1,), (1,)), ((), ())),
                           preferred_element_type=jnp.float32)


def _lrelu(x):
    return jnp.where(x >= 0, x, 0.2 * x)


def _mm_body(x_ref, w_ref, o_ref):
    o_ref[...] = jnp.dot(x_ref[...], w_ref[...],
                         preferred_element_type=jnp.float32)


def _mm(x, w):
    n, din = x.shape
    dout = w.shape[1]
    return pl.pallas_call(
        _mm_body,
        grid=(n // _BN,),
        in_specs=[pl.BlockSpec((_BN, din), lambda i: (i, 0)),
                  pl.BlockSpec((din, dout), lambda i: (0, 0))],
        out_specs=pl.BlockSpec((_BN, dout), lambda i: (i, 0)),
        out_shape=jax.ShapeDtypeStruct((n, dout), jnp.float32),
    )(x, w)


def _fuse_body(act, p_ref, w_ref, o_ref):
    a = p_ref[0] + p_ref[1]
    if act:
        a = _lrelu(a)
    o_ref[...] = jnp.dot(a, w_ref[...], preferred_element_type=jnp.float32)


def _fuse(p, w, act=True):
    _, n, din = p.shape
    dout = w.shape[1]
    return pl.pallas_call(
        functools.partial(_fuse_body, act),
        grid=(n // _BN,),
        in_specs=[pl.BlockSpec((2, _BN, din), lambda i: (0, i, 0)),
                  pl.BlockSpec((din, dout), lambda i: (0, 0))],
        out_specs=pl.BlockSpec((_BN, dout), lambda i: (i, 0)),
        out_shape=jax.ShapeDtypeStruct((n, dout), jnp.float32),
    )(p, w)


def _add_body(p_ref, o_ref):
    o_ref[...] = p_ref[0] + p_ref[1]


def _add(p):
    _, n, d = p.shape
    return pl.pallas_call(
        _add_body,
        grid=(n // _BN,),
        in_specs=[pl.BlockSpec((2, _BN, d), lambda i: (0, i, 0))],
        out_specs=pl.BlockSpec((_BN, d), lambda i: (i, 0)),
        out_shape=jax.ShapeDtypeStruct((n, d), jnp.float32),
    )(p)


def _addmm_body(p_ref, w_ref, z_ref, t_ref):
    z = p_ref[0] + p_ref[1]
    z_ref[...] = z
    t_ref[...] = jnp.dot(z, w_ref[...], preferred_element_type=jnp.float32)


def _addmm(p, w):
    _, n, din = p.shape
    dout = w.shape[1]
    return pl.pallas_call(
        _addmm_body,
        grid=(n // _BN,),
        in_specs=[pl.BlockSpec((2, _BN, din), lambda i: (0, i, 0)),
                  pl.BlockSpec((din, dout), lambda i: (0, 0))],
        out_specs=[pl.BlockSpec((_BN, din), lambda i: (i, 0)),
                   pl.BlockSpec((_BN, dout), lambda i: (i, 0))],
        out_shape=[jax.ShapeDtypeStruct((n, din), jnp.float32),
                   jax.ShapeDtypeStruct((n, dout), jnp.float32)],
    )(p, w)


_BADJ = 512


def _adj_body(zi_i, zh_i, zi_j, zh_j, o_ref):
    g1 = _dot_t(zi_i[...], zi_j[...])
    g2 = _dot_t(zh_i[...], zh_j[...])
    o_ref[...] = jax.nn.sigmoid(g1) + jax.nn.sigmoid(g2)


def _adj(zi, zh):
    n, dz = zi.shape
    dh = zh.shape[1]
    return pl.pallas_call(
        _adj_body,
        grid=(n // _BADJ, n // _BADJ),
        in_specs=[pl.BlockSpec((_BADJ, dz), lambda i, j: (i, 0)),
                  pl.BlockSpec((_BADJ, dh), lambda i, j: (i, 0)),
                  pl.BlockSpec((_BADJ, dz), lambda i, j: (j, 0)),
                  pl.BlockSpec((_BADJ, dh), lambda i, j: (j, 0))],
        out_specs=pl.BlockSpec((_BADJ, _BADJ), lambda i, j: (i, j)),
        out_shape=jax.ShapeDtypeStruct((n, n), jnp.float32),
    )(zi, zh, zi, zh)


def _st_body(z_ref, c_ref, o_ref):
    z = z_ref[...]
    c = c_ref[...]
    zz = jnp.sum(z * z, axis=1, keepdims=True)
    cc = jnp.sum(c * c, axis=1).reshape(1, -1)
    d = zz - 2.0 * _dot_t(z, c) + cc
    q = 1.0 / (1.0 + d)
    o_ref[...] = q / jnp.sum(q, axis=1, keepdims=True)


def _student_t(z, centers):
    n, dz = z.shape
    ncl = centers.shape[0]
    return pl.pallas_call(
        _st_body,
        grid=(1,),
        in_specs=[pl.BlockSpec((n, dz), lambda i: (0, 0)),
                  pl.BlockSpec((ncl, dz), lambda i: (0, 0))],
        out_specs=pl.BlockSpec((n, ncl), lambda i: (0, 0)),
        out_shape=jax.ShapeDtypeStruct((n, ncl), jnp.float32),
    )(z, centers)


def _view(x, idx, val, we1, we2, we3, wd1, wd2, wd3, centers):
    t = _mm(x, we1)
    p = _spmm(t, idx, val)
    t = _fuse(p, we2)
    p = _spmm(t, idx, val)
    t = _fuse(p, we3)
    p = _spmm(t, idx, val)
    z_igae = _add(p)
    p = _spmm(z_igae, idx, val)
    z1, t = _addmm(p, wd1)
    p = _spmm(t, idx, val)
    t = _fuse(p, wd2)
    p = _spmm(t, idx, val)
    t = _fuse(p, wd3)
    p = _spmm(t, idx, val)
    z_hat = _add(p)
    a_hat = _adj(z_igae, z_hat)
    qa = _student_t(z1, centers)
    qb = _student_t(z_igae, centers)
    return z_hat, a_hat, (qa, qb), z1


def kernel(x1, adj1_idx, adj1_val, x2, adj2_idx, adj2_val,
           w_e1_1, w_e1_2, w_e1_3, w_d1_1, w_d1_2, w_d1_3,
           w_e2_1, w_e2_2, w_e2_3, w_d2_1, w_d2_2, w_d2_3,
           centers1, centers2):
    z_hat1, a_hat1, Q1, z1 = _view(x1, adj1_idx, adj1_val,
                                   w_e1_1, w_e1_2, w_e1_3,
                                   w_d1_1, w_d1_2, w_d1_3, centers1)
    z_hat2, a_hat2, Q2, z2 = _view(x2, adj2_idx, adj2_val,
                                   w_e2_1, w_e2_2, w_e2_3,
                                   w_d2_1, w_d2_2, w_d2_3, centers2)
    return (z_hat1, a_hat1, z_hat2, a_hat2, Q1, Q2, z1, z2, (z1, z2))


# speedup vs baseline: 5.0278x; 1.6112x over previous
import functools

import jax
import jax.numpy as jnp
from jax import lax
from jax.experimental import pallas as pl
from jax.experimental.pallas import tpu as pltpu
from jax.experimental.pallas import tpu_sc as plsc

_N = 4096
_E = 65536
_NZ = 32
_NCL = 10

_NC = 2
_NS = 16
_LANES = 16
_NW = _NC * _NS
_EPW = _E // _NW
_RPS = _N // _NS
_ZR = 16


@functools.lru_cache(maxsize=None)
def _make_spmm(width):
    mesh = plsc.VectorSubcoreMesh(core_axis_name="c", subcore_axis_name="s")
    jw = width // _LANES
    _CB = min(128, 16384 // width)
    _TPC = _EPW // _CB

    def body(y_hbm, row_hbm, col_hbm, val_hbm, out_hbm,
             acc_sh, rb0, rb1, row_v, col_v, val_v, zbuf,
             gs0, gs1, ss0, ss1, isem):
        cid = lax.axis_index("c")
        sid = lax.axis_index("s")
        wid = sid * _NC + cid

        pltpu.async_copy(row_hbm.at[wid], row_v, isem)
        pltpu.async_copy(col_hbm.at[wid], col_v, isem)
        pltpu.async_copy(val_hbm.at[wid], val_v, isem)

        zero16 = jnp.zeros((_LANES,), jnp.float32)

        def zrow(r, carry):
            for j in range(jw):
                zbuf[r, pl.ds(j * _LANES, _LANES)] = zero16
            return carry

        lax.fori_loop(0, _ZR, zrow, 0)
        for rr in range(_RPS // _ZR):
            pltpu.sync_copy(zbuf, acc_sh.at[pl.ds(sid * _RPS + rr * _ZR, _ZR)])
        pltpu.make_async_copy(row_hbm.at[wid], row_v, isem).wait()
        pltpu.make_async_copy(col_hbm.at[wid], col_v, isem).wait()
        pltpu.make_async_copy(val_hbm.at[wid], val_v, isem).wait()
        plsc.subcore_barrier()

        def scale(rb, t):
            def grp(g, c2):
                vv = val_v[t, pl.ds(g * _LANES, _LANES)]
                for k in range(_LANES):
                    v = vv[k]
                    e = g * _LANES + k
                    for j in range(jw):
                        sl = pl.ds(j * _LANES, _LANES)
                        rb[e, sl] = rb[e, sl] * v
                return c2

            lax.fori_loop(0, _CB // _LANES, grp, 0)

        pltpu.async_copy(y_hbm.at[col_v.at[0]], rb0, gs0)

        def pair(p, carry):
            t0 = 2 * p
            t1 = t0 + 1

            @pl.when(p > 0)
            def _():
                pltpu.make_async_copy(rb1, acc_sh.at[row_v.at[t1]], ss1).wait()

            pltpu.async_copy(y_hbm.at[col_v.at[t1]], rb1, gs1)
            pltpu.make_async_copy(y_hbm.at[col_v.at[t0]], rb0, gs0).wait()
            scale(rb0, t0)
            pltpu.async_copy(rb0, acc_sh.at[row_v.at[t0]], ss0, add=True)
            pltpu.make_async_copy(y_hbm.at[col_v.at[t1]], rb1, gs1).wait()
            scale(rb1, t1)
            pltpu.async_copy(rb1, acc_sh.at[row_v.at[t1]], ss1, add=True)
            pltpu.make_async_copy(rb0, acc_sh.at[row_v.at[t0]], ss0).wait()

            @pl.when(p < _TPC // 2 - 1)
            def _():
                pltpu.async_copy(y_hbm.at[col_v.at[t0 + 2]], rb0, gs0)

            return carry

        lax.fori_loop(0, _TPC // 2, pair, 0)
        pltpu.make_async_copy(rb1, acc_sh.at[row_v.at[_TPC - 1]], ss1).wait()

        plsc.subcore_barrier()
        pltpu.sync_copy(acc_sh.at[pl.ds(sid * _RPS, _RPS)],
                        out_hbm.at[cid, pl.ds(sid * _RPS, _RPS)])

    return pl.kernel(
        body,
        out_type=jax.ShapeDtypeStruct((_NC, _N, width), jnp.float32),
        mesh=mesh,
        compiler_params=pltpu.CompilerParams(use_tc_tiling_on_sc=False),
        scratch_types=[
            pltpu.VMEM_SHARED((_N, width), jnp.float32),
            pltpu.VMEM((_CB, width), jnp.float32),
            pltpu.VMEM((_CB, width), jnp.float32),
            pltpu.VMEM((_TPC, _CB), jnp.int32),
            pltpu.VMEM((_TPC, _CB), jnp.int32),
            pltpu.VMEM((_TPC, _CB), jnp.float32),
            pltpu.VMEM((_ZR, width), jnp.float32),
            pltpu.SemaphoreType.DMA,
            pltpu.SemaphoreType.DMA,
            pltpu.SemaphoreType.DMA,
            pltpu.SemaphoreType.DMA,
            pltpu.SemaphoreType.DMA,
        ],
    )


def _spmm(y, idx, val):
    cb = min(128, 16384 // y.shape[1])
    tpc = _EPW // cb
    row = idx[0].reshape(_NW, tpc, cb)
    col = idx[1].reshape(_NW, tpc, cb)
    v3 = val.reshape(_NW, tpc, cb)
    return _make_spmm(y.shape[1])(y, row, col, v3)


_BN = 1024


def _dot_t(a, b):
    return lax.dot_general(a, b, (((1,), (1,)), ((), ())),
                           preferred_element_type=jnp.float32)


def _lrelu(x):
    return jnp.where(x >= 0, x, 0.2 * x)


def _mm_body(x_ref, w_ref, o_ref):
    o_ref[...] = jnp.dot(x_ref[...], w_ref[...],
                         preferred_element_type=jnp.float32)


def _mm(x, w):
    n, din = x.shape
    dout = w.shape[1]
    return pl.pallas_call(
        _mm_body,
        grid=(n // _BN,),
        in_specs=[pl.BlockSpec((_BN, din), lambda i: (i, 0)),
                  pl.BlockSpec((din, dout), lambda i: (0, 0))],
        out_specs=pl.BlockSpec((_BN, dout), lambda i: (i, 0)),
        out_shape=jax.ShapeDtypeStruct((n, dout), jnp.float32),
    )(x, w)


def _fuse_body(act, p_ref, w_ref, o_ref):
    a = p_ref[0] + p_ref[1]
    if act:
        a = _lrelu(a)
    o_ref[...] = jnp.dot(a, w_ref[...], preferred_element_type=jnp.float32)


def _fuse(p, w, act=True):
    _, n, din = p.shape
    dout = w.shape[1]
    return pl.pallas_call(
        functools.partial(_fuse_body, act),
        grid=(n // _BN,),
        in_specs=[pl.BlockSpec((2, _BN, din), lambda i: (0, i, 0)),
                  pl.BlockSpec((din, dout), lambda i: (0, 0))],
        out_specs=pl.BlockSpec((_BN, dout), lambda i: (i, 0)),
        out_shape=jax.ShapeDtypeStruct((n, dout), jnp.float32),
    )(p, w)


def _add_body(p_ref, o_ref):
    o_ref[...] = p_ref[0] + p_ref[1]


def _add(p):
    _, n, d = p.shape
    return pl.pallas_call(
        _add_body,
        grid=(n // _BN,),
        in_specs=[pl.BlockSpec((2, _BN, d), lambda i: (0, i, 0))],
        out_specs=pl.BlockSpec((_BN, d), lambda i: (i, 0)),
        out_shape=jax.ShapeDtypeStruct((n, d), jnp.float32),
    )(p)


def _addmm_body(p_ref, w_ref, z_ref, t_ref):
    z = p_ref[0] + p_ref[1]
    z_ref[...] = z
    t_ref[...] = jnp.dot(z, w_ref[...], preferred_element_type=jnp.float32)


def _addmm(p, w):
    _, n, din = p.shape
    dout = w.shape[1]
    return pl.pallas_call(
        _addmm_body,
        grid=(n // _BN,),
        in_specs=[pl.BlockSpec((2, _BN, din), lambda i: (0, i, 0)),
                  pl.BlockSpec((din, dout), lambda i: (0, 0))],
        out_specs=[pl.BlockSpec((_BN, din), lambda i: (i, 0)),
                   pl.BlockSpec((_BN, dout), lambda i: (i, 0))],
        out_shape=[jax.ShapeDtypeStruct((n, din), jnp.float32),
                   jax.ShapeDtypeStruct((n, dout), jnp.float32)],
    )(p, w)


_BADJ = 512


def _adj_body(zi_i, zh_i, zi_j, zh_j, o_ref):
    g1 = _dot_t(zi_i[...], zi_j[...])
    g2 = _dot_t(zh_i[...], zh_j[...])
    o_ref[...] = jax.nn.sigmoid(g1) + jax.nn.sigmoid(g2)


def _adj(zi, zh):
    n, dz = zi.shape
    dh = zh.shape[1]
    return pl.pallas_call(
        _adj_body,
        grid=(n // _BADJ, n // _BADJ),
        in_specs=[pl.BlockSpec((_BADJ, dz), lambda i, j: (i, 0)),
                  pl.BlockSpec((_BADJ, dh), lambda i, j: (i, 0)),
                  pl.BlockSpec((_BADJ, dz), lambda i, j: (j, 0)),
                  pl.BlockSpec((_BADJ, dh), lambda i, j: (j, 0))],
        out_specs=pl.BlockSpec((_BADJ, _BADJ), lambda i, j: (i, j)),
        out_shape=jax.ShapeDtypeStruct((n, n), jnp.float32),
    )(zi, zh, zi, zh)


def _st_body(z_ref, c_ref, o_ref):
    z = z_ref[...]
    c = c_ref[...]
    zz = jnp.sum(z * z, axis=1, keepdims=True)
    cc = jnp.sum(c * c, axis=1).reshape(1, -1)
    d = zz - 2.0 * _dot_t(z, c) + cc
    q = 1.0 / (1.0 + d)
    o_ref[...] = q / jnp.sum(q, axis=1, keepdims=True)


def _student_t(z, centers):
    n, dz = z.shape
    ncl = centers.shape[0]
    return pl.pallas_call(
        _st_body,
        grid=(1,),
        in_specs=[pl.BlockSpec((n, dz), lambda i: (0, 0)),
                  pl.BlockSpec((ncl, dz), lambda i: (0, 0))],
        out_specs=pl.BlockSpec((n, ncl), lambda i: (0, 0)),
        out_shape=jax.ShapeDtypeStruct((n, ncl), jnp.float32),
    )(z, centers)


def _view(x, idx, val, we1, we2, we3, wd1, wd2, wd3, centers):
    t = _mm(x, we1)
    p = _spmm(t, idx, val)
    t = _fuse(p, we2)
    p = _spmm(t, idx, val)
    t = _fuse(p, we3)
    p = _spmm(t, idx, val)
    z_igae = _add(p)
    p = _spmm(z_igae, idx, val)
    z1, t = _addmm(p, wd1)
    p = _spmm(t, idx, val)
    t = _fuse(p, wd2)
    p = _spmm(t, idx, val)
    t = _fuse(p, wd3)
    p = _spmm(t, idx, val)
    z_hat = _add(p)
    a_hat = _adj(z_igae, z_hat)
    qa = _student_t(z1, centers)
    qb = _student_t(z_igae, centers)
    return z_hat, a_hat, (qa, qb), z1


def kernel(x1, adj1_idx, adj1_val, x2, adj2_idx, adj2_val,
           w_e1_1, w_e1_2, w_e1_3, w_d1_1, w_d1_2, w_d1_3,
           w_e2_1, w_e2_2, w_e2_3, w_d2_1, w_d2_2, w_d2_3,
           centers1, centers2):
    z_hat1, a_hat1, Q1, z1 = _view(x1, adj1_idx, adj1_val,
                                   w_e1_1, w_e1_2, w_e1_3,
                                   w_d1_1, w_d1_2, w_d1_3, centers1)
    z_hat2, a_hat2, Q2, z2 = _view(x2, adj2_idx, adj2_val,
                                   w_e2_1, w_e2_2, w_e2_3,
                                   w_d2_1, w_d2_2, w_d2_3, centers2)
    return (z_hat1, a_hat1, z_hat2, a_hat2, Q1, Q2, z1, z2, (z1, z2))


# speedup vs baseline: 5.5476x; 1.1034x over previous
import functools

import jax
import jax.numpy as jnp
from jax import lax
from jax.experimental import pallas as pl
from jax.experimental.pallas import tpu as pltpu
from jax.experimental.pallas import tpu_sc as plsc

_N = 4096
_E = 65536
_NZ = 32
_NCL = 10

_NC = 2
_NS = 16
_LANES = 16
_NW = _NC * _NS
_EPW = _E // _NW
_RPS = _N // _NS
_ZR = 64


@functools.lru_cache(maxsize=None)
def _make_spmm(width):
    mesh = plsc.VectorSubcoreMesh(core_axis_name="c", subcore_axis_name="s")
    jw = width // _LANES
    _CB = 32 if width > 128 else 128
    _TPC = _EPW // _CB
    _NQ = _TPC // 4

    def body(y_hbm, row_hbm, col_hbm, val_hbm, out_hbm,
             acc_sh, rb0, rb1, rb2, rb3, row_v, col_v, val_v, zbuf,
             gs0, gs1, gs2, gs3, ss0, ss1, ss2, ss3, isem, zsem):
        cid = lax.axis_index("c")
        sid = lax.axis_index("s")
        wid = sid * _NC + cid
        rbs = (rb0, rb1, rb2, rb3)
        gss = (gs0, gs1, gs2, gs3)
        sss = (ss0, ss1, ss2, ss3)

        pltpu.async_copy(row_hbm.at[wid], row_v, isem)
        pltpu.async_copy(col_hbm.at[wid], col_v, isem)
        pltpu.async_copy(val_hbm.at[wid], val_v, isem)

        zero16 = jnp.zeros((_LANES,), jnp.float32)

        def zrow(r, carry):
            for j in range(jw):
                zbuf[r, pl.ds(j * _LANES, _LANES)] = zero16
            return carry

        lax.fori_loop(0, _ZR, zrow, 0)
        for rr in range(_RPS // _ZR):
            pltpu.async_copy(zbuf, acc_sh.at[pl.ds(sid * _RPS + rr * _ZR, _ZR)],
                             zsem)
        for rr in range(_RPS // _ZR):
            pltpu.make_async_copy(
                zbuf, acc_sh.at[pl.ds(sid * _RPS + rr * _ZR, _ZR)], zsem).wait()
        pltpu.make_async_copy(row_hbm.at[wid], row_v, isem).wait()
        pltpu.make_async_copy(col_hbm.at[wid], col_v, isem).wait()
        pltpu.make_async_copy(val_hbm.at[wid], val_v, isem).wait()
        plsc.subcore_barrier()

        def scale(rb, t):
            def grp(g, c2):
                vv = val_v[t, pl.ds(g * _LANES, _LANES)]
                for k in range(_LANES):
                    v = vv[k]
                    e = g * _LANES + k
                    for j in range(jw):
                        sl = pl.ds(j * _LANES, _LANES)
                        rb[e, sl] = rb[e, sl] * v
                return c2

            lax.fori_loop(0, _CB // _LANES, grp, 0)

        pltpu.async_copy(y_hbm.at[col_v.at[0]], rb0, gs0)
        pltpu.async_copy(y_hbm.at[col_v.at[1]], rb1, gs1)

        def quad(q, carry):
            for i in range(4):
                t = 4 * q + i
                pltpu.make_async_copy(y_hbm.at[col_v.at[t]], rbs[i],
                                      gss[i]).wait()
                scale(rbs[i], t)
                pltpu.async_copy(rbs[i], acc_sh.at[row_v.at[t]], sss[i],
                                 add=True)
                i2 = (i + 2) % 4
                if i < 2:
                    @pl.when(q > 0)
                    def _():
                        pltpu.make_async_copy(
                            rbs[i2], acc_sh.at[row_v.at[t]], sss[i2]).wait()

                    pltpu.async_copy(y_hbm.at[col_v.at[t + 2]], rbs[i2],
                                     gss[i2])
                else:
                    pltpu.make_async_copy(
                        rbs[i2], acc_sh.at[row_v.at[t]], sss[i2]).wait()

                    @pl.when(q < _NQ - 1)
                    def _():
                        pltpu.async_copy(y_hbm.at[col_v.at[t + 2]], rbs[i2],
                                         gss[i2])

            return carry

        lax.fori_loop(0, _NQ, quad, 0)
        pltpu.make_async_copy(rb2, acc_sh.at[row_v.at[_TPC - 2]], ss2).wait()
        pltpu.make_async_copy(rb3, acc_sh.at[row_v.at[_TPC - 1]], ss3).wait()

        plsc.subcore_barrier()
        pltpu.sync_copy(acc_sh.at[pl.ds(sid * _RPS, _RPS)],
                        out_hbm.at[cid, pl.ds(sid * _RPS, _RPS)])

    return pl.kernel(
        body,
        out_type=jax.ShapeDtypeStruct((_NC, _N, width), jnp.float32),
        mesh=mesh,
        compiler_params=pltpu.CompilerParams(use_tc_tiling_on_sc=False),
        scratch_types=[
            pltpu.VMEM_SHARED((_N, width), jnp.float32),
            pltpu.VMEM((_CB, width), jnp.float32),
            pltpu.VMEM((_CB, width), jnp.float32),
            pltpu.VMEM((_CB, width), jnp.float32),
            pltpu.VMEM((_CB, width), jnp.float32),
            pltpu.VMEM((_TPC, _CB), jnp.int32),
            pltpu.VMEM((_TPC, _CB), jnp.int32),
            pltpu.VMEM((_TPC, _CB), jnp.float32),
            pltpu.VMEM((_ZR, width), jnp.float32),
            pltpu.SemaphoreType.DMA,
            pltpu.SemaphoreType.DMA,
            pltpu.SemaphoreType.DMA,
            pltpu.SemaphoreType.DMA,
            pltpu.SemaphoreType.DMA,
            pltpu.SemaphoreType.DMA,
            pltpu.SemaphoreType.DMA,
            pltpu.SemaphoreType.DMA,
            pltpu.SemaphoreType.DMA,
            pltpu.SemaphoreType.DMA,
        ],
    )


def _spmm(y, idx, val):
    cb = 32 if y.shape[1] > 128 else 128
    tpc = _EPW // cb
    row = idx[0].reshape(_NW, tpc, cb)
    col = idx[1].reshape(_NW, tpc, cb)
    v3 = val.reshape(_NW, tpc, cb)
    return _make_spmm(y.shape[1])(y, row, col, v3)


_BN = 1024


def _dot_t(a, b):
    return lax.dot_general(a, b, (((1,), (1,)), ((), ())),
                           preferred_element_type=jnp.float32)


def _lrelu(x):
    return jnp.where(x >= 0, x, 0.2 * x)


def _mm_body(x_ref, w_ref, o_ref):
    o_ref[...] = jnp.dot(x_ref[...], w_ref[...],
                         preferred_element_type=jnp.float32)


def _mm(x, w):
    n, din = x.shape
    dout = w.shape[1]
    return pl.pallas_call(
        _mm_body,
        grid=(n // _BN,),
        in_specs=[pl.BlockSpec((_BN, din), lambda i: (i, 0)),
                  pl.BlockSpec((din, dout), lambda i: (0, 0))],
        out_specs=pl.BlockSpec((_BN, dout), lambda i: (i, 0)),
        out_shape=jax.ShapeDtypeStruct((n, dout), jnp.float32),
    )(x, w)


def _fuse_body(act, p_ref, w_ref, o_ref):
    a = p_ref[0] + p_ref[1]
    if act:
        a = _lrelu(a)
    o_ref[...] = jnp.dot(a, w_ref[...], preferred_element_type=jnp.float32)


def _fuse(p, w, act=True):
    _, n, din = p.shape
    dout = w.shape[1]
    return pl.pallas_call(
        functools.partial(_fuse_body, act),
        grid=(n // _BN,),
        in_specs=[pl.BlockSpec((2, _BN, din), lambda i: (0, i, 0)),
                  pl.BlockSpec((din, dout), lambda i: (0, 0))],
        out_specs=pl.BlockSpec((_BN, dout), lambda i: (i, 0)),
        out_shape=jax.ShapeDtypeStruct((n, dout), jnp.float32),
    )(p, w)


def _add_body(p_ref, o_ref):
    o_ref[...] = p_ref[0] + p_ref[1]


def _add(p):
    _, n, d = p.shape
    return pl.pallas_call(
        _add_body,
        grid=(n // _BN,),
        in_specs=[pl.BlockSpec((2, _BN, d), lambda i: (0, i, 0))],
        out_specs=pl.BlockSpec((_BN, d), lambda i: (i, 0)),
        out_shape=jax.ShapeDtypeStruct((n, d), jnp.float32),
    )(p)


def _addmm_body(p_ref, w_ref, z_ref, t_ref):
    z = p_ref[0] + p_ref[1]
    z_ref[...] = z
    t_ref[...] = jnp.dot(z, w_ref[...], preferred_element_type=jnp.float32)


def _addmm(p, w):
    _, n, din = p.shape
    dout = w.shape[1]
    return pl.pallas_call(
        _addmm_body,
        grid=(n // _BN,),
        in_specs=[pl.BlockSpec((2, _BN, din), lambda i: (0, i, 0)),
                  pl.BlockSpec((din, dout), lambda i: (0, 0))],
        out_specs=[pl.BlockSpec((_BN, din), lambda i: (i, 0)),
                   pl.BlockSpec((_BN, dout), lambda i: (i, 0))],
        out_shape=[jax.ShapeDtypeStruct((n, din), jnp.float32),
                   jax.ShapeDtypeStruct((n, dout), jnp.float32)],
    )(p, w)


_BADJ = 512


def _adj_body(zi_i, zh_i, zi_j, zh_j, o_ref):
    g1 = _dot_t(zi_i[...], zi_j[...])
    g2 = _dot_t(zh_i[...], zh_j[...])
    o_ref[...] = jax.nn.sigmoid(g1) + jax.nn.sigmoid(g2)


def _adj(zi, zh):
    n, dz = zi.shape
    dh = zh.shape[1]
    return pl.pallas_call(
        _adj_body,
        grid=(n // _BADJ, n // _BADJ),
        in_specs=[pl.BlockSpec((_BADJ, dz), lambda i, j: (i, 0)),
                  pl.BlockSpec((_BADJ, dh), lambda i, j: (i, 0)),
                  pl.BlockSpec((_BADJ, dz), lambda i, j: (j, 0)),
                  pl.BlockSpec((_BADJ, dh), lambda i, j: (j, 0))],
        out_specs=pl.BlockSpec((_BADJ, _BADJ), lambda i, j: (i, j)),
        out_shape=jax.ShapeDtypeStruct((n, n), jnp.float32),
    )(zi, zh, zi, zh)


def _st_body(z_ref, c_ref, o_ref):
    z = z_ref[...]
    c = c_ref[...]
    zz = jnp.sum(z * z, axis=1, keepdims=True)
    cc = jnp.sum(c * c, axis=1).reshape(1, -1)
    d = zz - 2.0 * _dot_t(z, c) + cc
    q = 1.0 / (1.0 + d)
    o_ref[...] = q / jnp.sum(q, axis=1, keepdims=True)


def _student_t(z, centers):
    n, dz = z.shape
    ncl = centers.shape[0]
    return pl.pallas_call(
        _st_body,
        grid=(1,),
        in_specs=[pl.BlockSpec((n, dz), lambda i: (0, 0)),
                  pl.BlockSpec((ncl, dz), lambda i: (0, 0))],
        out_specs=pl.BlockSpec((n, ncl), lambda i: (0, 0)),
        out_shape=jax.ShapeDtypeStruct((n, ncl), jnp.float32),
    )(z, centers)


def _view(x, idx, val, we1, we2, we3, wd1, wd2, wd3, centers):
    t = _mm(x, we1)
    p = _spmm(t, idx, val)
    t = _fuse(p, we2)
    p = _spmm(t, idx, val)
    t = _fuse(p, we3)
    p = _spmm(t, idx, val)
    z_igae = _add(p)
    p = _spmm(z_igae, idx, val)
    z1, t = _addmm(p, wd1)
    p = _spmm(t, idx, val)
    t = _fuse(p, wd2)
    p = _spmm(t, idx, val)
    t = _fuse(p, wd3)
    p = _spmm(t, idx, val)
    z_hat = _add(p)
    a_hat = _adj(z_igae, z_hat)
    qa = _student_t(z1, centers)
    qb = _student_t(z_igae, centers)
    return z_hat, a_hat, (qa, qb), z1


def kernel(x1, adj1_idx, adj1_val, x2, adj2_idx, adj2_val,
           w_e1_1, w_e1_2, w_e1_3, w_d1_1, w_d1_2, w_d1_3,
           w_e2_1, w_e2_2, w_e2_3, w_d2_1, w_d2_2, w_d2_3,
           centers1, centers2):
    z_hat1, a_hat1, Q1, z1 = _view(x1, adj1_idx, adj1_val,
                                   w_e1_1, w_e1_2, w_e1_3,
                                   w_d1_1, w_d1_2, w_d1_3, centers1)
    z_hat2, a_hat2, Q2, z2 = _view(x2, adj2_idx, adj2_val,
                                   w_e2_1, w_e2_2, w_e2_3,
                                   w_d2_1, w_d2_2, w_d2_3, centers2)
    return (z_hat1, a_hat1, z_hat2, a_hat2, Q1, Q2, z1, z2, (z1, z2))


# speedup vs baseline: 5.6450x; 1.0176x over previous
import functools

import jax
import jax.numpy as jnp
from jax import lax
from jax.experimental import pallas as pl
from jax.experimental.pallas import tpu as pltpu
from jax.experimental.pallas import tpu_sc as plsc

_N = 4096
_E = 65536
_NZ = 32
_NCL = 10

_NC = 2
_NS = 16
_LANES = 16
_NW = _NC * _NS
_EPW = _E // _NW
_RPS = _N // _NS
_ZR = 64


@functools.lru_cache(maxsize=None)
def _make_spmm(width):
    mesh = plsc.VectorSubcoreMesh(core_axis_name="c", subcore_axis_name="s")
    jw = width // _LANES
    _CB = 32 if width > 128 else 128
    _TPC = _EPW // _CB
    _NQ = _TPC // 4

    def body(y_hbm, row_hbm, col_hbm, val_hbm, out_hbm,
             acc_sh, rb0, rb1, rb2, rb3, row_v, col_v, val_v, zbuf,
             gs0, gs1, gs2, gs3, ss0, ss1, ss2, ss3, isem, zsem):
        cid = lax.axis_index("c")
        sid = lax.axis_index("s")
        wid = sid * _NC + cid
        rbs = (rb0, rb1, rb2, rb3)
        gss = (gs0, gs1, gs2, gs3)
        sss = (ss0, ss1, ss2, ss3)

        pltpu.async_copy(row_hbm.at[wid], row_v, isem)
        pltpu.async_copy(col_hbm.at[wid], col_v, isem)
        pltpu.async_copy(val_hbm.at[wid], val_v, isem)

        zero16 = jnp.zeros((_LANES,), jnp.float32)

        def zrow(r, carry):
            for j in range(jw):
                zbuf[r, pl.ds(j * _LANES, _LANES)] = zero16
            return carry

        lax.fori_loop(0, _ZR, zrow, 0)
        for rr in range(_RPS // _ZR):
            pltpu.async_copy(zbuf, acc_sh.at[pl.ds(sid * _RPS + rr * _ZR, _ZR)],
                             zsem)
        for rr in range(_RPS // _ZR):
            pltpu.make_async_copy(
                zbuf, acc_sh.at[pl.ds(sid * _RPS + rr * _ZR, _ZR)], zsem).wait()
        pltpu.make_async_copy(row_hbm.at[wid], row_v, isem).wait()
        pltpu.make_async_copy(col_hbm.at[wid], col_v, isem).wait()
        pltpu.make_async_copy(val_hbm.at[wid], val_v, isem).wait()
        plsc.subcore_barrier()

        def scale(rb, t):
            def grp(g, c2):
                vv = val_v[t, pl.ds(g * _LANES, _LANES)]
                for k in range(_LANES):
                    v = vv[k]
                    e = g * _LANES + k
                    for j in range(jw):
                        sl = pl.ds(j * _LANES, _LANES)
                        rb[e, sl] = rb[e, sl] * v
                return c2

            lax.fori_loop(0, _CB // _LANES, grp, 0)

        pltpu.async_copy(y_hbm.at[col_v.at[0]], rb0, gs0)
        pltpu.async_copy(y_hbm.at[col_v.at[1]], rb1, gs1)

        def quad(q, carry):
            for i in range(4):
                t = 4 * q + i
                pltpu.make_async_copy(y_hbm.at[col_v.at[t]], rbs[i],
                                      gss[i]).wait()
                scale(rbs[i], t)
                pltpu.async_copy(rbs[i], acc_sh.at[row_v.at[t]], sss[i],
                                 add=True)
                i2 = (i + 2) % 4
                if i < 2:
                    @pl.when(q > 0)
                    def _():
                        pltpu.make_async_copy(
                            rbs[i2], acc_sh.at[row_v.at[t]], sss[i2]).wait()

                    pltpu.async_copy(y_hbm.at[col_v.at[t + 2]], rbs[i2],
                                     gss[i2])
                else:
                    pltpu.make_async_copy(
                        rbs[i2], acc_sh.at[row_v.at[t]], sss[i2]).wait()

                    @pl.when(q < _NQ - 1)
                    def _():
                        pltpu.async_copy(y_hbm.at[col_v.at[t + 2]], rbs[i2],
                                         gss[i2])

            return carry

        lax.fori_loop(0, _NQ, quad, 0)
        pltpu.make_async_copy(rb2, acc_sh.at[row_v.at[_TPC - 2]], ss2).wait()
        pltpu.make_async_copy(rb3, acc_sh.at[row_v.at[_TPC - 1]], ss3).wait()

        plsc.subcore_barrier()
        pltpu.sync_copy(acc_sh.at[pl.ds(sid * _RPS, _RPS)],
                        out_hbm.at[cid, pl.ds(sid * _RPS, _RPS)])

    return pl.kernel(
        body,
        out_type=jax.ShapeDtypeStruct((_NC, _N, width), jnp.float32),
        mesh=mesh,
        compiler_params=pltpu.CompilerParams(use_tc_tiling_on_sc=False),
        scratch_types=[
            pltpu.VMEM_SHARED((_N, width), jnp.float32),
            pltpu.VMEM((_CB, width), jnp.float32),
            pltpu.VMEM((_CB, width), jnp.float32),
            pltpu.VMEM((_CB, width), jnp.float32),
            pltpu.VMEM((_CB, width), jnp.float32),
            pltpu.VMEM((_TPC, _CB), jnp.int32),
            pltpu.VMEM((_TPC, _CB), jnp.int32),
            pltpu.VMEM((_TPC, _CB), jnp.float32),
            pltpu.VMEM((_ZR, width), jnp.float32),
            pltpu.SemaphoreType.DMA,
            pltpu.SemaphoreType.DMA,
            pltpu.SemaphoreType.DMA,
            pltpu.SemaphoreType.DMA,
            pltpu.SemaphoreType.DMA,
            pltpu.SemaphoreType.DMA,
            pltpu.SemaphoreType.DMA,
            pltpu.SemaphoreType.DMA,
            pltpu.SemaphoreType.DMA,
            pltpu.SemaphoreType.DMA,
        ],
    )


def _spmm(y, idx, val):
    cb = 32 if y.shape[1] > 128 else 128
    tpc = _EPW // cb
    row = idx[0].reshape(_NW, tpc, cb)
    col = idx[1].reshape(_NW, tpc, cb)
    v3 = val.reshape(_NW, tpc, cb)
    return _make_spmm(y.shape[1])(y, row, col, v3)


_EPW2 = _E // _NS


@functools.lru_cache(maxsize=None)
def _make_spmm2(width):
    mesh = plsc.VectorSubcoreMesh(core_axis_name="c", subcore_axis_name="s")
    jw = width // _LANES
    _CB = 64 if width >= 128 else 128
    _TPC = _EPW2 // _CB
    _NQ = _TPC // 4

    def body(y_hbm, row_hbm, col_hbm, val_hbm, out_hbm,
             acc_sh, rb0, rb1, rb2, rb3, row_v, col_v, val_v, zbuf,
             gs0, gs1, gs2, gs3, ss0, ss1, ss2, ss3, isem, zsem):
        cid = lax.axis_index("c")
        sid = lax.axis_index("s")
        rbs = (rb0, rb1, rb2, rb3)
        gss = (gs0, gs1, gs2, gs3)
        sss = (ss0, ss1, ss2, ss3)
        ysrc = y_hbm.at[cid]

        pltpu.async_copy(row_hbm.at[cid, sid], row_v, isem)
        pltpu.async_copy(col_hbm.at[cid, sid], col_v, isem)
        pltpu.async_copy(val_hbm.at[cid, sid], val_v, isem)

        zero16 = jnp.zeros((_LANES,), jnp.float32)

        def zrow(r, carry):
            for j in range(jw):
                zbuf[r, pl.ds(j * _LANES, _LANES)] = zero16
            return carry

        lax.fori_loop(0, _ZR // 2, zrow, 0)
        nzc = _RPS // (_ZR // 2)
        for rr in range(nzc):
            pltpu.async_copy(
                zbuf, acc_sh.at[pl.ds(sid * _RPS + rr * (_ZR // 2), _ZR // 2)],
                zsem)
        for rr in range(nzc):
            pltpu.make_async_copy(
                zbuf, acc_sh.at[pl.ds(sid * _RPS + rr * (_ZR // 2), _ZR // 2)],
                zsem).wait()
        pltpu.make_async_copy(row_hbm.at[cid, sid], row_v, isem).wait()
        pltpu.make_async_copy(col_hbm.at[cid, sid], col_v, isem).wait()
        pltpu.make_async_copy(val_hbm.at[cid, sid], val_v, isem).wait()
        plsc.subcore_barrier()

        def scale(rb, t):
            def grp(g, c2):
                vv = val_v[t, pl.ds(g * _LANES, _LANES)]
                for k in range(_LANES):
                    v = vv[k]
                    e = g * _LANES + k
                    for j in range(jw):
                        sl = pl.ds(j * _LANES, _LANES)
                        rb[e, sl] = rb[e, sl] * v
                return c2

            lax.fori_loop(0, _CB // _LANES, grp, 0)

        pltpu.async_copy(ysrc.at[col_v.at[0]], rb0, gs0)
        pltpu.async_copy(ysrc.at[col_v.at[1]], rb1, gs1)

        def quad(q, carry):
            for i in range(4):
                t = 4 * q + i
                pltpu.make_async_copy(ysrc.at[col_v.at[t]], rbs[i],
                                      gss[i]).wait()
                scale(rbs[i], t)
                pltpu.async_copy(rbs[i], acc_sh.at[row_v.at[t]], sss[i],
                                 add=True)
                i2 = (i + 2) % 4
                if i < 2:
                    @pl.when(q > 0)
                    def _():
                        pltpu.make_async_copy(
                            rbs[i2], acc_sh.at[row_v.at[t]], sss[i2]).wait()

                    pltpu.async_copy(ysrc.at[col_v.at[t + 2]], rbs[i2],
                                     gss[i2])
                else:
                    pltpu.make_async_copy(
                        rbs[i2], acc_sh.at[row_v.at[t]], sss[i2]).wait()

                    @pl.when(q < _NQ - 1)
                    def _():
                        pltpu.async_copy(ysrc.at[col_v.at[t + 2]], rbs[i2],
                                         gss[i2])

            return carry

        lax.fori_loop(0, _NQ, quad, 0)
        pltpu.make_async_copy(rb2, acc_sh.at[row_v.at[_TPC - 2]], ss2).wait()
        pltpu.make_async_copy(rb3, acc_sh.at[row_v.at[_TPC - 1]], ss3).wait()

        plsc.subcore_barrier()
        pltpu.sync_copy(acc_sh.at[pl.ds(sid * _RPS, _RPS)],
                        out_hbm.at[cid, pl.ds(sid * _RPS, _RPS)])

    return pl.kernel(
        body,
        out_type=jax.ShapeDtypeStruct((2, _N, width), jnp.float32),
        mesh=mesh,
        compiler_params=pltpu.CompilerParams(use_tc_tiling_on_sc=False),
        scratch_types=[
            pltpu.VMEM_SHARED((_N, width), jnp.float32),
            pltpu.VMEM((_CB, width), jnp.float32),
            pltpu.VMEM((_CB, width), jnp.float32),
            pltpu.VMEM((_CB, width), jnp.float32),
            pltpu.VMEM((_CB, width), jnp.float32),
            pltpu.VMEM((_TPC, _CB), jnp.int32),
            pltpu.VMEM((_TPC, _CB), jnp.int32),
            pltpu.VMEM((_TPC, _CB), jnp.float32),
            pltpu.VMEM((_ZR // 2, width), jnp.float32),
            pltpu.SemaphoreType.DMA,
            pltpu.SemaphoreType.DMA,
            pltpu.SemaphoreType.DMA,
            pltpu.SemaphoreType.DMA,
            pltpu.SemaphoreType.DMA,
            pltpu.SemaphoreType.DMA,
            pltpu.SemaphoreType.DMA,
            pltpu.SemaphoreType.DMA,
            pltpu.SemaphoreType.DMA,
            pltpu.SemaphoreType.DMA,
        ],
    )


def _edges2(idx1, val1, idx2, val2, width):
    cb = 64 if width >= 128 else 128
    tpc = _EPW2 // cb
    row = jnp.stack([idx1[0].reshape(_NS, tpc, cb),
                     idx2[0].reshape(_NS, tpc, cb)])
    col = jnp.stack([idx1[1].reshape(_NS, tpc, cb),
                     idx2[1].reshape(_NS, tpc, cb)])
    v = jnp.stack([val1.reshape(_NS, tpc, cb), val2.reshape(_NS, tpc, cb)])
    return row, col, v


def _spmm2(y_s, edges):
    row, col, v = edges
    return _make_spmm2(y_s.shape[2])(y_s, row, col, v)


_BN = 1024


def _dot_t(a, b):
    return lax.dot_general(a, b, (((1,), (1,)), ((), ())),
                           preferred_element_type=jnp.float32)


def _lrelu(x):
    return jnp.where(x >= 0, x, 0.2 * x)


def _mm_body(x_ref, w_ref, o_ref):
    o_ref[...] = jnp.dot(x_ref[...], w_ref[...],
                         preferred_element_type=jnp.float32)


def _mm(x, w):
    n, din = x.shape
    dout = w.shape[1]
    return pl.pallas_call(
        _mm_body,
        grid=(n // _BN,),
        in_specs=[pl.BlockSpec((_BN, din), lambda i: (i, 0)),
                  pl.BlockSpec((din, dout), lambda i: (0, 0))],
        out_specs=pl.BlockSpec((_BN, dout), lambda i: (i, 0)),
        out_shape=jax.ShapeDtypeStruct((n, dout), jnp.float32),
    )(x, w)


def _fuse_body(act, p_ref, w_ref, o_ref):
    a = p_ref[0] + p_ref[1]
    if act:
        a = _lrelu(a)
    o_ref[...] = jnp.dot(a, w_ref[...], preferred_element_type=jnp.float32)


def _fuse(p, w, act=True):
    _, n, din = p.shape
    dout = w.shape[1]
    return pl.pallas_call(
        functools.partial(_fuse_body, act),
        grid=(n // _BN,),
        in_specs=[pl.BlockSpec((2, _BN, din), lambda i: (0, i, 0)),
                  pl.BlockSpec((din, dout), lambda i: (0, 0))],
        out_specs=pl.BlockSpec((_BN, dout), lambda i: (i, 0)),
        out_shape=jax.ShapeDtypeStruct((n, dout), jnp.float32),
    )(p, w)


def _fuse2_body(act, s_ref, w_ref, o_ref):
    a = s_ref[0]
    if act:
        a = _lrelu(a)
    o_ref[0] = jnp.dot(a, w_ref[0], preferred_element_type=jnp.float32)


def _fuse2(s, w_s, act=True):
    _, n, din = s.shape
    dout = w_s.shape[2]
    return pl.pallas_call(
        functools.partial(_fuse2_body, act),
        grid=(2, n // _BN),
        in_specs=[pl.BlockSpec((1, _BN, din), lambda v, i: (v, i, 0)),
                  pl.BlockSpec((1, din, dout), lambda v, i: (v, 0, 0))],
        out_specs=pl.BlockSpec((1, _BN, dout), lambda v, i: (v, i, 0)),
        out_shape=jax.ShapeDtypeStruct((2, n, dout), jnp.float32),
    )(s, w_s)


def _fuse1_body(act, s_ref, w_ref, o_ref):
    a = s_ref[0]
    if act:
        a = _lrelu(a)
    o_ref[...] = jnp.dot(a, w_ref[...], preferred_element_type=jnp.float32)


def _fuse1(s, view, w, act=True):
    _, n, din = s.shape
    dout = w.shape[1]
    return pl.pallas_call(
        functools.partial(_fuse1_body, act),
        grid=(n // _BN,),
        in_specs=[pl.BlockSpec((1, _BN, din), lambda i: (view, i, 0)),
                  pl.BlockSpec((din, dout), lambda i: (0, 0))],
        out_specs=pl.BlockSpec((_BN, dout), lambda i: (i, 0)),
        out_shape=jax.ShapeDtypeStruct((n, dout), jnp.float32),
    )(s, w)


def _add_body(p_ref, o_ref):
    o_ref[...] = p_ref[0] + p_ref[1]


def _add(p):
    _, n, d = p.shape
    return pl.pallas_call(
        _add_body,
        grid=(n // _BN,),
        in_specs=[pl.BlockSpec((2, _BN, d), lambda i: (0, i, 0))],
        out_specs=pl.BlockSpec((_BN, d), lambda i: (i, 0)),
        out_shape=jax.ShapeDtypeStruct((n, d), jnp.float32),
    )(p)


def _addmm_body(p_ref, w_ref, z_ref, t_ref):
    z = p_ref[0] + p_ref[1]
    z_ref[...] = z
    t_ref[...] = jnp.dot(z, w_ref[...], preferred_element_type=jnp.float32)


def _addmm(p, w):
    _, n, din = p.shape
    dout = w.shape[1]
    return pl.pallas_call(
        _addmm_body,
        grid=(n // _BN,),
        in_specs=[pl.BlockSpec((2, _BN, din), lambda i: (0, i, 0)),
                  pl.BlockSpec((din, dout), lambda i: (0, 0))],
        out_specs=[pl.BlockSpec((_BN, din), lambda i: (i, 0)),
                   pl.BlockSpec((_BN, dout), lambda i: (i, 0))],
        out_shape=[jax.ShapeDtypeStruct((n, din), jnp.float32),
                   jax.ShapeDtypeStruct((n, dout), jnp.float32)],
    )(p, w)


_BADJ = 512


def _adj_body(zi_i, zh_i, zi_j, zh_j, o_ref):
    g1 = _dot_t(zi_i[...], zi_j[...])
    g2 = _dot_t(zh_i[...], zh_j[...])
    o_ref[...] = jax.nn.sigmoid(g1) + jax.nn.sigmoid(g2)


def _adj(zi, zh):
    n, dz = zi.shape
    dh = zh.shape[1]
    return pl.pallas_call(
        _adj_body,
        grid=(n // _BADJ, n // _BADJ),
        in_specs=[pl.BlockSpec((_BADJ, dz), lambda i, j: (i, 0)),
                  pl.BlockSpec((_BADJ, dh), lambda i, j: (i, 0)),
                  pl.BlockSpec((_BADJ, dz), lambda i, j: (j, 0)),
                  pl.BlockSpec((_BADJ, dh), lambda i, j: (j, 0))],
        out_specs=pl.BlockSpec((_BADJ, _BADJ), lambda i, j: (i, j)),
        out_shape=jax.ShapeDtypeStruct((n, n), jnp.float32),
    )(zi, zh, zi, zh)


def _st_body(z_ref, c_ref, o_ref):
    z = z_ref[...]
    c = c_ref[...]
    zz = jnp.sum(z * z, axis=1, keepdims=True)
    cc = jnp.sum(c * c, axis=1).reshape(1, -1)
    d = zz - 2.0 * _dot_t(z, c) + cc
    q = 1.0 / (1.0 + d)
    o_ref[...] = q / jnp.sum(q, axis=1, keepdims=True)


def _student_t(z, centers):
    n, dz = z.shape
    ncl = centers.shape[0]
    return pl.pallas_call(
        _st_body,
        grid=(1,),
        in_specs=[pl.BlockSpec((n, dz), lambda i: (0, 0)),
                  pl.BlockSpec((ncl, dz), lambda i: (0, 0))],
        out_specs=pl.BlockSpec((n, ncl), lambda i: (0, 0)),
        out_shape=jax.ShapeDtypeStruct((n, ncl), jnp.float32),
    )(z, centers)


def kernel(x1, adj1_idx, adj1_val, x2, adj2_idx, adj2_val,
           w_e1_1, w_e1_2, w_e1_3, w_d1_1, w_d1_2, w_d1_3,
           w_e2_1, w_e2_2, w_e2_3, w_d2_1, w_d2_2, w_d2_3,
           centers1, centers2):
    ewide = _edges2(adj1_idx, adj1_val, adj2_idx, adj2_val, 128)
    enarrow = _edges2(adj1_idx, adj1_val, adj2_idx, adj2_val, 64)

    ys = jnp.stack([_mm(x1, w_e1_1), _mm(x2, w_e2_1)])
    s = _spmm2(ys, ewide)
    ys = _fuse2(s, jnp.stack([w_e1_2, w_e2_2]))
    s = _spmm2(ys, enarrow)
    ys = _fuse2(s, jnp.stack([w_e1_3, w_e2_3]))
    zi_s = _spmm2(ys, enarrow)
    z1_s = _spmm2(zi_s, enarrow)
    ys = _fuse2(z1_s, jnp.stack([w_d1_1, w_d2_1]), act=False)
    s = _spmm2(ys, enarrow)
    ys = _fuse2(s, jnp.stack([w_d1_2, w_d2_2]))
    s = _spmm2(ys, ewide)
    t1 = _fuse1(s, 0, w_d1_3)
    t2 = _fuse1(s, 1, w_d2_3)
    z_hat1 = _add(_spmm(t1, adj1_idx, adj1_val))
    z_hat2 = _add(_spmm(t2, adj2_idx, adj2_val))

    zi1, zi2 = zi_s[0], zi_s[1]
    z1, z2 = z1_s[0], z1_s[1]
    a_hat1 = _adj(zi1, z_hat1)
    a_hat2 = _adj(zi2, z_hat2)
    Q1 = (_student_t(z1, centers1), _student_t(zi1, centers1))
    Q2 = (_student_t(z2, centers2), _student_t(zi2, centers2))
    return (z_hat1, a_hat1, z_hat2, a_hat2, Q1, Q2, z1, z2, (z1, z2))


# speedup vs baseline: 5.7136x; 1.0122x over previous
import functools

import jax
import jax.numpy as jnp
from jax import lax
from jax.experimental import pallas as pl
from jax.experimental.pallas import tpu as pltpu
from jax.experimental.pallas import tpu_sc as plsc

_N = 4096
_E = 65536
_NZ = 32
_NCL = 10

_NC = 2
_NS = 16
_LANES = 16
_NW = _NC * _NS
_EPW = _E // _NW
_RPS = _N // _NS
_ZR = 64


@functools.lru_cache(maxsize=None)
def _make_spmm(width):
    mesh = plsc.VectorSubcoreMesh(core_axis_name="c", subcore_axis_name="s")
    jw = width // _LANES
    _CB = 32 if width > 128 else 128
    _TPC = _EPW // _CB
    _NQ = _TPC // 4

    def body(y_hbm, row_hbm, col_hbm, val_hbm, out_hbm,
             acc_sh, rb0, rb1, rb2, rb3, row_v, col_v, val_v, zbuf,
             gs0, gs1, gs2, gs3, ss0, ss1, ss2, ss3, isem, zsem):
        cid = lax.axis_index("c")
        sid = lax.axis_index("s")
        wid = sid * _NC + cid
        rbs = (rb0, rb1, rb2, rb3)
        gss = (gs0, gs1, gs2, gs3)
        sss = (ss0, ss1, ss2, ss3)

        pltpu.async_copy(row_hbm.at[wid], row_v, isem)
        pltpu.async_copy(col_hbm.at[wid], col_v, isem)
        pltpu.async_copy(val_hbm.at[wid], val_v, isem)

        zero16 = jnp.zeros((_LANES,), jnp.float32)

        def zrow(r, carry):
            for j in range(jw):
                zbuf[r, pl.ds(j * _LANES, _LANES)] = zero16
            return carry

        lax.fori_loop(0, _ZR, zrow, 0)
        for rr in range(_RPS // _ZR):
            pltpu.async_copy(zbuf, acc_sh.at[pl.ds(sid * _RPS + rr * _ZR, _ZR)],
                             zsem)
        for rr in range(_RPS // _ZR):
            pltpu.make_async_copy(
                zbuf, acc_sh.at[pl.ds(sid * _RPS + rr * _ZR, _ZR)], zsem).wait()
        pltpu.make_async_copy(row_hbm.at[wid], row_v, isem).wait()
        pltpu.make_async_copy(col_hbm.at[wid], col_v, isem).wait()
        pltpu.make_async_copy(val_hbm.at[wid], val_v, isem).wait()
        plsc.subcore_barrier()

        def scale(rb, t):
            def grp(g, c2):
                vv = val_v[t, pl.ds(g * _LANES, _LANES)]
                for k in range(_LANES):
                    v = vv[k]
                    e = g * _LANES + k
                    for j in range(jw):
                        sl = pl.ds(j * _LANES, _LANES)
                        rb[e, sl] = rb[e, sl] * v
                return c2

            lax.fori_loop(0, _CB // _LANES, grp, 0)

        pltpu.async_copy(y_hbm.at[col_v.at[0]], rb0, gs0)
        pltpu.async_copy(y_hbm.at[col_v.at[1]], rb1, gs1)

        def quad(q, carry):
            for i in range(4):
                t = 4 * q + i
                pltpu.make_async_copy(y_hbm.at[col_v.at[t]], rbs[i],
                                      gss[i]).wait()
                scale(rbs[i], t)
                pltpu.async_copy(rbs[i], acc_sh.at[row_v.at[t]], sss[i],
                                 add=True)
                i2 = (i + 2) % 4
                if i < 2:
                    @pl.when(q > 0)
                    def _():
                        pltpu.make_async_copy(
                            rbs[i2], acc_sh.at[row_v.at[t]], sss[i2]).wait()

                    pltpu.async_copy(y_hbm.at[col_v.at[t + 2]], rbs[i2],
                                     gss[i2])
                else:
                    pltpu.make_async_copy(
                        rbs[i2], acc_sh.at[row_v.at[t]], sss[i2]).wait()

                    @pl.when(q < _NQ - 1)
                    def _():
                        pltpu.async_copy(y_hbm.at[col_v.at[t + 2]], rbs[i2],
                                         gss[i2])

            return carry

        lax.fori_loop(0, _NQ, quad, 0)
        pltpu.make_async_copy(rb2, acc_sh.at[row_v.at[_TPC - 2]], ss2).wait()
        pltpu.make_async_copy(rb3, acc_sh.at[row_v.at[_TPC - 1]], ss3).wait()

        plsc.subcore_barrier()
        pltpu.sync_copy(acc_sh.at[pl.ds(sid * _RPS, _RPS)],
                        out_hbm.at[cid, pl.ds(sid * _RPS, _RPS)])

    return pl.kernel(
        body,
        out_type=jax.ShapeDtypeStruct((_NC, _N, width), jnp.float32),
        mesh=mesh,
        compiler_params=pltpu.CompilerParams(use_tc_tiling_on_sc=False),
        scratch_types=[
            pltpu.VMEM_SHARED((_N, width), jnp.float32),
            pltpu.VMEM((_CB, width), jnp.float32),
            pltpu.VMEM((_CB, width), jnp.float32),
            pltpu.VMEM((_CB, width), jnp.float32),
            pltpu.VMEM((_CB, width), jnp.float32),
            pltpu.VMEM((_TPC, _CB), jnp.int32),
            pltpu.VMEM((_TPC, _CB), jnp.int32),
            pltpu.VMEM((_TPC, _CB), jnp.float32),
            pltpu.VMEM((_ZR, width), jnp.float32),
            pltpu.SemaphoreType.DMA,
            pltpu.SemaphoreType.DMA,
            pltpu.SemaphoreType.DMA,
            pltpu.SemaphoreType.DMA,
            pltpu.SemaphoreType.DMA,
            pltpu.SemaphoreType.DMA,
            pltpu.SemaphoreType.DMA,
            pltpu.SemaphoreType.DMA,
            pltpu.SemaphoreType.DMA,
            pltpu.SemaphoreType.DMA,
        ],
    )


def _spmm(y, idx, val):
    cb = 32 if y.shape[1] > 128 else 128
    tpc = _EPW // cb
    row = idx[0].reshape(_NW, tpc, cb)
    col = idx[1].reshape(_NW, tpc, cb)
    v3 = val.reshape(_NW, tpc, cb)
    return _make_spmm(y.shape[1])(y, row, col, v3)


_EPW2 = _E // _NS


@functools.lru_cache(maxsize=None)
def _make_spmm2(width):
    mesh = plsc.VectorSubcoreMesh(core_axis_name="c", subcore_axis_name="s")
    jw = width // _LANES
    _CB = 64 if width >= 128 else 128
    _TPC = _EPW2 // _CB
    _NQ = _TPC // 4

    def body(y_hbm, row_hbm, col_hbm, val_hbm, out_hbm,
             acc_sh, rb0, rb1, rb2, rb3, row_v, col_v, val_v, zbuf,
             gs0, gs1, gs2, gs3, ss0, ss1, ss2, ss3, isem, zsem):
        cid = lax.axis_index("c")
        sid = lax.axis_index("s")
        rbs = (rb0, rb1, rb2, rb3)
        gss = (gs0, gs1, gs2, gs3)
        sss = (ss0, ss1, ss2, ss3)
        ysrc = y_hbm.at[cid]

        pltpu.async_copy(row_hbm.at[cid, sid], row_v, isem)
        pltpu.async_copy(col_hbm.at[cid, sid], col_v, isem)
        pltpu.async_copy(val_hbm.at[cid, sid], val_v, isem)

        zero16 = jnp.zeros((_LANES,), jnp.float32)

        def zrow(r, carry):
            for j in range(jw):
                zbuf[r, pl.ds(j * _LANES, _LANES)] = zero16
            return carry

        lax.fori_loop(0, _ZR // 2, zrow, 0)
        nzc = _RPS // (_ZR // 2)
        for rr in range(nzc):
            pltpu.async_copy(
                zbuf, acc_sh.at[pl.ds(sid * _RPS + rr * (_ZR // 2), _ZR // 2)],
                zsem)
        for rr in range(nzc):
            pltpu.make_async_copy(
                zbuf, acc_sh.at[pl.ds(sid * _RPS + rr * (_ZR // 2), _ZR // 2)],
                zsem).wait()
        pltpu.make_async_copy(row_hbm.at[cid, sid], row_v, isem).wait()
        pltpu.make_async_copy(col_hbm.at[cid, sid], col_v, isem).wait()
        pltpu.make_async_copy(val_hbm.at[cid, sid], val_v, isem).wait()
        plsc.subcore_barrier()

        def scale(rb, t):
            def grp(g, c2):
                vv = val_v[t, pl.ds(g * _LANES, _LANES)]
                for k in range(_LANES):
                    v = vv[k]
                    e = g * _LANES + k
                    for j in range(jw):
                        sl = pl.ds(j * _LANES, _LANES)
                        rb[e, sl] = rb[e, sl] * v
                return c2

            lax.fori_loop(0, _CB // _LANES, grp, 0)

        pltpu.async_copy(ysrc.at[col_v.at[0]], rb0, gs0)
        pltpu.async_copy(ysrc.at[col_v.at[1]], rb1, gs1)

        def quad(q, carry):
            for i in range(4):
                t = 4 * q + i
                pltpu.make_async_copy(ysrc.at[col_v.at[t]], rbs[i],
                                      gss[i]).wait()
                scale(rbs[i], t)
                pltpu.async_copy(rbs[i], acc_sh.at[row_v.at[t]], sss[i],
                                 add=True)
                i2 = (i + 2) % 4
                if i < 2:
                    @pl.when(q > 0)
                    def _():
                        pltpu.make_async_copy(
                            rbs[i2], acc_sh.at[row_v.at[t]], sss[i2]).wait()

                    pltpu.async_copy(ysrc.at[col_v.at[t + 2]], rbs[i2],
                                     gss[i2])
                else:
                    pltpu.make_async_copy(
                        rbs[i2], acc_sh.at[row_v.at[t]], sss[i2]).wait()

                    @pl.when(q < _NQ - 1)
                    def _():
                        pltpu.async_copy(ysrc.at[col_v.at[t + 2]], rbs[i2],
                                         gss[i2])

            return carry

        lax.fori_loop(0, _NQ, quad, 0)
        pltpu.make_async_copy(rb2, acc_sh.at[row_v.at[_TPC - 2]], ss2).wait()
        pltpu.make_async_copy(rb3, acc_sh.at[row_v.at[_TPC - 1]], ss3).wait()

        plsc.subcore_barrier()
        pltpu.sync_copy(acc_sh.at[pl.ds(sid * _RPS, _RPS)],
                        out_hbm.at[cid, pl.ds(sid * _RPS, _RPS)])

    return pl.kernel(
        body,
        out_type=jax.ShapeDtypeStruct((2, _N, width), jnp.float32),
        mesh=mesh,
        compiler_params=pltpu.CompilerParams(use_tc_tiling_on_sc=False),
        scratch_types=[
            pltpu.VMEM_SHARED((_N, width), jnp.float32),
            pltpu.VMEM((_CB, width), jnp.float32),
            pltpu.VMEM((_CB, width), jnp.float32),
            pltpu.VMEM((_CB, width), jnp.float32),
            pltpu.VMEM((_CB, width), jnp.float32),
            pltpu.VMEM((_TPC, _CB), jnp.int32),
            pltpu.VMEM((_TPC, _CB), jnp.int32),
            pltpu.VMEM((_TPC, _CB), jnp.float32),
            pltpu.VMEM((_ZR // 2, width), jnp.float32),
            pltpu.SemaphoreType.DMA,
            pltpu.SemaphoreType.DMA,
            pltpu.SemaphoreType.DMA,
            pltpu.SemaphoreType.DMA,
            pltpu.SemaphoreType.DMA,
            pltpu.SemaphoreType.DMA,
            pltpu.SemaphoreType.DMA,
            pltpu.SemaphoreType.DMA,
            pltpu.SemaphoreType.DMA,
            pltpu.SemaphoreType.DMA,
        ],
    )


@functools.lru_cache(maxsize=None)
def _make_spmm2x(width):
    mesh = plsc.VectorSubcoreMesh(core_axis_name="c", subcore_axis_name="s")
    jw = width // _LANES
    _CB = 64 if width >= 128 else 128
    _TPC = _EPW2 // _CB
    _NQ = _TPC // 4

    def body(y_hbm, row_hbm, col_hbm, val_hbm, outa_hbm, outb_hbm,
             acca_sh, accb_sh, rb0, rb1, rb2, rb3, row_v, col_v, val_v, zbuf,
             gs0, gs1, gs2, gs3, ss0, ss1, ss2, ss3, isem, zsem, esem):
        cid = lax.axis_index("c")
        sid = lax.axis_index("s")
        rbs = (rb0, rb1, rb2, rb3)
        gss = (gs0, gs1, gs2, gs3)
        sss = (ss0, ss1, ss2, ss3)

        pltpu.async_copy(row_hbm.at[cid, sid], row_v, isem)
        pltpu.async_copy(col_hbm.at[cid, sid], col_v, isem)
        pltpu.async_copy(val_hbm.at[cid, sid], val_v, isem)

        zero16 = jnp.zeros((_LANES,), jnp.float32)

        def zrow(r, carry):
            for j in range(jw):
                zbuf[r, pl.ds(j * _LANES, _LANES)] = zero16
            return carry

        lax.fori_loop(0, _ZR // 2, zrow, 0)
        nzc = _RPS // (_ZR // 2)
        for acc in (acca_sh, accb_sh):
            for rr in range(nzc):
                pltpu.async_copy(
                    zbuf,
                    acc.at[pl.ds(sid * _RPS + rr * (_ZR // 2), _ZR // 2)],
                    zsem)
        for acc in (acca_sh, accb_sh):
            for rr in range(nzc):
                pltpu.make_async_copy(
                    zbuf,
                    acc.at[pl.ds(sid * _RPS + rr * (_ZR // 2), _ZR // 2)],
                    zsem).wait()
        pltpu.make_async_copy(row_hbm.at[cid, sid], row_v, isem).wait()
        pltpu.make_async_copy(col_hbm.at[cid, sid], col_v, isem).wait()
        pltpu.make_async_copy(val_hbm.at[cid, sid], val_v, isem).wait()
        plsc.subcore_barrier()

        def scale(rb, t):
            def grp(g, c2):
                vv = val_v[t, pl.ds(g * _LANES, _LANES)]
                for k in range(_LANES):
                    v = vv[k]
                    e = g * _LANES + k
                    for j in range(jw):
                        sl = pl.ds(j * _LANES, _LANES)
                        rb[e, sl] = rb[e, sl] * v
                return c2

            lax.fori_loop(0, _CB // _LANES, grp, 0)

        def hop(ysrc, acc):
            pltpu.async_copy(ysrc.at[col_v.at[0]], rb0, gs0)
            pltpu.async_copy(ysrc.at[col_v.at[1]], rb1, gs1)

            def quad(q, carry):
                for i in range(4):
                    t = 4 * q + i
                    pltpu.make_async_copy(ysrc.at[col_v.at[t]], rbs[i],
                                          gss[i]).wait()
                    scale(rbs[i], t)
                    pltpu.async_copy(rbs[i], acc.at[row_v.at[t]], sss[i],
                                     add=True)
                    i2 = (i + 2) % 4
                    if i < 2:
                        @pl.when(q > 0)
                        def _():
                            pltpu.make_async_copy(
                                rbs[i2], acc.at[row_v.at[t]], sss[i2]).wait()

                        pltpu.async_copy(ysrc.at[col_v.at[t + 2]], rbs[i2],
                                         gss[i2])
                    else:
                        pltpu.make_async_copy(
                            rbs[i2], acc.at[row_v.at[t]], sss[i2]).wait()

                        @pl.when(q < _NQ - 1)
                        def _():
                            pltpu.async_copy(ysrc.at[col_v.at[t + 2]],
                                             rbs[i2], gss[i2])

                return carry

            lax.fori_loop(0, _NQ, quad, 0)
            pltpu.make_async_copy(rb2, acc.at[row_v.at[_TPC - 2]], ss2).wait()
            pltpu.make_async_copy(rb3, acc.at[row_v.at[_TPC - 1]], ss3).wait()

        hop(y_hbm.at[cid], acca_sh)
        plsc.subcore_barrier()
        pltpu.async_copy(acca_sh.at[pl.ds(sid * _RPS, _RPS)],
                         outa_hbm.at[cid, pl.ds(sid * _RPS, _RPS)], esem)
        hop(acca_sh, accb_sh)
        pltpu.make_async_copy(acca_sh.at[pl.ds(sid * _RPS, _RPS)],
                              outa_hbm.at[cid, pl.ds(sid * _RPS, _RPS)],
                              esem).wait()
        plsc.subcore_barrier()
        pltpu.sync_copy(accb_sh.at[pl.ds(sid * _RPS, _RPS)],
                        outb_hbm.at[cid, pl.ds(sid * _RPS, _RPS)])

    return pl.kernel(
        body,
        out_type=[jax.ShapeDtypeStruct((2, _N, width), jnp.float32),
                  jax.ShapeDtypeStruct((2, _N, width), jnp.float32)],
        mesh=mesh,
        compiler_params=pltpu.CompilerParams(use_tc_tiling_on_sc=False),
        scratch_types=[
            pltpu.VMEM_SHARED((_N, width), jnp.float32),
            pltpu.VMEM_SHARED((_N, width), jnp.float32),
            pltpu.VMEM((_CB, width), jnp.float32),
            pltpu.VMEM((_CB, width), jnp.float32),
            pltpu.VMEM((_CB, width), jnp.float32),
            pltpu.VMEM((_CB, width), jnp.float32),
            pltpu.VMEM((_TPC, _CB), jnp.int32),
            pltpu.VMEM((_TPC, _CB), jnp.int32),
            pltpu.VMEM((_TPC, _CB), jnp.float32),
            pltpu.VMEM((_ZR // 2, width), jnp.float32),
            pltpu.SemaphoreType.DMA,
            pltpu.SemaphoreType.DMA,
            pltpu.SemaphoreType.DMA,
            pltpu.SemaphoreType.DMA,
            pltpu.SemaphoreType.DMA,
            pltpu.SemaphoreType.DMA,
            pltpu.SemaphoreType.DMA,
            pltpu.SemaphoreType.DMA,
            pltpu.SemaphoreType.DMA,
            pltpu.SemaphoreType.DMA,
            pltpu.SemaphoreType.DMA,
        ],
    )


def _spmm2x(y_s, edges):
    row, col, v = edges
    return _make_spmm2x(y_s.shape[2])(y_s, row, col, v)


def _edges2(idx1, val1, idx2, val2, width):
    cb = 64 if width >= 128 else 128
    tpc = _EPW2 // cb
    row = jnp.stack([idx1[0].reshape(_NS, tpc, cb),
                     idx2[0].reshape(_NS, tpc, cb)])
    col = jnp.stack([idx1[1].reshape(_NS, tpc, cb),
                     idx2[1].reshape(_NS, tpc, cb)])
    v = jnp.stack([val1.reshape(_NS, tpc, cb), val2.reshape(_NS, tpc, cb)])
    return row, col, v


def _spmm2(y_s, edges):
    row, col, v = edges
    return _make_spmm2(y_s.shape[2])(y_s, row, col, v)


_BN = 1024


def _dot_t(a, b):
    return lax.dot_general(a, b, (((1,), (1,)), ((), ())),
                           preferred_element_type=jnp.float32)


def _lrelu(x):
    return jnp.where(x >= 0, x, 0.2 * x)


def _mm_body(x_ref, w_ref, o_ref):
    o_ref[...] = jnp.dot(x_ref[...], w_ref[...],
                         preferred_element_type=jnp.float32)


def _mm(x, w):
    n, din = x.shape
    dout = w.shape[1]
    return pl.pallas_call(
        _mm_body,
        grid=(n // _BN,),
        in_specs=[pl.BlockSpec((_BN, din), lambda i: (i, 0)),
                  pl.BlockSpec((din, dout), lambda i: (0, 0))],
        out_specs=pl.BlockSpec((_BN, dout), lambda i: (i, 0)),
        out_shape=jax.ShapeDtypeStruct((n, dout), jnp.float32),
    )(x, w)


def _fuse_body(act, p_ref, w_ref, o_ref):
    a = p_ref[0] + p_ref[1]
    if act:
        a = _lrelu(a)
    o_ref[...] = jnp.dot(a, w_ref[...], preferred_element_type=jnp.float32)


def _fuse(p, w, act=True):
    _, n, din = p.shape
    dout = w.shape[1]
    return pl.pallas_call(
        functools.partial(_fuse_body, act),
        grid=(n // _BN,),
        in_specs=[pl.BlockSpec((2, _BN, din), lambda i: (0, i, 0)),
                  pl.BlockSpec((din, dout), lambda i: (0, 0))],
        out_specs=pl.BlockSpec((_BN, dout), lambda i: (i, 0)),
        out_shape=jax.ShapeDtypeStruct((n, dout), jnp.float32),
    )(p, w)


def _fuse2_body(act, s_ref, w_ref, o_ref):
    a = s_ref[0]
    if act:
        a = _lrelu(a)
    o_ref[0] = jnp.dot(a, w_ref[0], preferred_element_type=jnp.float32)


def _fuse2(s, w_s, act=True):
    _, n, din = s.shape
    dout = w_s.shape[2]
    return pl.pallas_call(
        functools.partial(_fuse2_body, act),
        grid=(2, n // _BN),
        in_specs=[pl.BlockSpec((1, _BN, din), lambda v, i: (v, i, 0)),
                  pl.BlockSpec((1, din, dout), lambda v, i: (v, 0, 0))],
        out_specs=pl.BlockSpec((1, _BN, dout), lambda v, i: (v, i, 0)),
        out_shape=jax.ShapeDtypeStruct((2, n, dout), jnp.float32),
    )(s, w_s)


def _fuse1_body(act, s_ref, w_ref, o_ref):
    a = s_ref[0]
    if act:
        a = _lrelu(a)
    o_ref[...] = jnp.dot(a, w_ref[...], preferred_element_type=jnp.float32)


def _fuse1(s, view, w, act=True):
    _, n, din = s.shape
    dout = w.shape[1]
    return pl.pallas_call(
        functools.partial(_fuse1_body, act),
        grid=(n // _BN,),
        in_specs=[pl.BlockSpec((1, _BN, din), lambda i: (view, i, 0)),
                  pl.BlockSpec((din, dout), lambda i: (0, 0))],
        out_specs=pl.BlockSpec((_BN, dout), lambda i: (i, 0)),
        out_shape=jax.ShapeDtypeStruct((n, dout), jnp.float32),
    )(s, w)


def _add_body(p_ref, o_ref):
    o_ref[...] = p_ref[0] + p_ref[1]


def _add(p):
    _, n, d = p.shape
    return pl.pallas_call(
        _add_body,
        grid=(n // _BN,),
        in_specs=[pl.BlockSpec((2, _BN, d), lambda i: (0, i, 0))],
        out_specs=pl.BlockSpec((_BN, d), lambda i: (i, 0)),
        out_shape=jax.ShapeDtypeStruct((n, d), jnp.float32),
    )(p)


def _addmm_body(p_ref, w_ref, z_ref, t_ref):
    z = p_ref[0] + p_ref[1]
    z_ref[...] = z
    t_ref[...] = jnp.dot(z, w_ref[...], preferred_element_type=jnp.float32)


def _addmm(p, w):
    _, n, din = p.shape
    dout = w.shape[1]
    return pl.pallas_call(
        _addmm_body,
        grid=(n // _BN,),
        in_specs=[pl.BlockSpec((2, _BN, din), lambda i: (0, i, 0)),
                  pl.BlockSpec((din, dout), lambda i: (0, 0))],
        out_specs=[pl.BlockSpec((_BN, din), lambda i: (i, 0)),
                   pl.BlockSpec((_BN, dout), lambda i: (i, 0))],
        out_shape=[jax.ShapeDtypeStruct((n, din), jnp.float32),
                   jax.ShapeDtypeStruct((n, dout), jnp.float32)],
    )(p, w)


_BADJ = 512


def _adj_body(zi_i, zh_i, zi_j, zh_j, o_ref):
    g1 = _dot_t(zi_i[...], zi_j[...])
    g2 = _dot_t(zh_i[...], zh_j[...])
    o_ref[...] = jax.nn.sigmoid(g1) + jax.nn.sigmoid(g2)


def _adj(zi, zh):
    n, dz = zi.shape
    dh = zh.shape[1]
    return pl.pallas_call(
        _adj_body,
        grid=(n // _BADJ, n // _BADJ),
        in_specs=[pl.BlockSpec((_BADJ, dz), lambda i, j: (i, 0)),
                  pl.BlockSpec((_BADJ, dh), lambda i, j: (i, 0)),
                  pl.BlockSpec((_BADJ, dz), lambda i, j: (j, 0)),
                  pl.BlockSpec((_BADJ, dh), lambda i, j: (j, 0))],
        out_specs=pl.BlockSpec((_BADJ, _BADJ), lambda i, j: (i, j)),
        out_shape=jax.ShapeDtypeStruct((n, n), jnp.float32),
    )(zi, zh, zi, zh)


def _st_body(z_ref, c_ref, o_ref):
    z = z_ref[...]
    c = c_ref[...]
    zz = jnp.sum(z * z, axis=1, keepdims=True)
    cc = jnp.sum(c * c, axis=1).reshape(1, -1)
    d = zz - 2.0 * _dot_t(z, c) + cc
    q = 1.0 / (1.0 + d)
    o_ref[...] = q / jnp.sum(q, axis=1, keepdims=True)


def _student_t(z, centers):
    n, dz = z.shape
    ncl = centers.shape[0]
    return pl.pallas_call(
        _st_body,
        grid=(1,),
        in_specs=[pl.BlockSpec((n, dz), lambda i: (0, 0)),
                  pl.BlockSpec((ncl, dz), lambda i: (0, 0))],
        out_specs=pl.BlockSpec((n, ncl), lambda i: (0, 0)),
        out_shape=jax.ShapeDtypeStruct((n, ncl), jnp.float32),
    )(z, centers)


def kernel(x1, adj1_idx, adj1_val, x2, adj2_idx, adj2_val,
           w_e1_1, w_e1_2, w_e1_3, w_d1_1, w_d1_2, w_d1_3,
           w_e2_1, w_e2_2, w_e2_3, w_d2_1, w_d2_2, w_d2_3,
           centers1, centers2):
    ewide = _edges2(adj1_idx, adj1_val, adj2_idx, adj2_val, 128)
    enarrow = _edges2(adj1_idx, adj1_val, adj2_idx, adj2_val, 64)

    ys = jnp.stack([_mm(x1, w_e1_1), _mm(x2, w_e2_1)])
    s = _spmm2(ys, ewide)
    ys = _fuse2(s, jnp.stack([w_e1_2, w_e2_2]))
    s = _spmm2(ys, enarrow)
    ys = _fuse2(s, jnp.stack([w_e1_3, w_e2_3]))
    zi_s, z1_s = _spmm2x(ys, enarrow)
    ys = _fuse2(z1_s, jnp.stack([w_d1_1, w_d2_1]), act=False)
    s = _spmm2(ys, enarrow)
    ys = _fuse2(s, jnp.stack([w_d1_2, w_d2_2]))
    s = _spmm2(ys, ewide)
    t1 = _fuse1(s, 0, w_d1_3)
    t2 = _fuse1(s, 1, w_d2_3)
    zi1, zi2 = zi_s[0], zi_s[1]
    z1, z2 = z1_s[0], z1_s[1]
    z_hat1 = _add(_spmm(t1, adj1_idx, adj1_val))
    a_hat1 = _adj(zi1, z_hat1)
    z_hat2 = _add(_spmm(t2, adj2_idx, adj2_val))
    a_hat2 = _adj(zi2, z_hat2)
    Q1 = (_student_t(z1, centers1), _student_t(zi1, centers1))
    Q2 = (_student_t(z2, centers2), _student_t(zi2, centers2))
    return (z_hat1, a_hat1, z_hat2, a_hat2, Q1, Q2, z1, z2, (z1, z2))


# speedup vs baseline: 6.3096x; 1.1043x over previous
import functools

import jax
import jax.numpy as jnp
from jax import lax
from jax.experimental import pallas as pl
from jax.experimental.pallas import tpu as pltpu
from jax.experimental.pallas import tpu_sc as plsc

_N = 4096
_E = 65536
_NZ = 32
_NCL = 10

_NC = 2
_NS = 16
_LANES = 16
_NW = _NC * _NS
_EPW = _E // _NW
_RPS = _N // _NS
_ZR = 64


@functools.lru_cache(maxsize=None)
def _make_spmm(width):
    mesh = plsc.VectorSubcoreMesh(core_axis_name="c", subcore_axis_name="s")
    jw = width // _LANES
    _CB = 32 if width > 128 else 128
    _TPC = _EPW // _CB
    _NQ = _TPC // 4

    def body(y_hbm, row_hbm, col_hbm, val_hbm, out_hbm,
             acc_sh, rb0, rb1, rb2, rb3, row_v, col_v, val_v, zbuf,
             gs0, gs1, gs2, gs3, ss0, ss1, ss2, ss3, isem, zsem):
        cid = lax.axis_index("c")
        sid = lax.axis_index("s")
        wid = sid * _NC + cid
        rbs = (rb0, rb1, rb2, rb3)
        gss = (gs0, gs1, gs2, gs3)
        sss = (ss0, ss1, ss2, ss3)

        pltpu.async_copy(row_hbm.at[wid], row_v, isem)
        pltpu.async_copy(col_hbm.at[wid], col_v, isem)
        pltpu.async_copy(val_hbm.at[wid], val_v, isem)

        zero16 = jnp.zeros((_LANES,), jnp.float32)

        def zrow(r, carry):
            for j in range(jw):
                zbuf[r, pl.ds(j * _LANES, _LANES)] = zero16
            return carry

        lax.fori_loop(0, _ZR, zrow, 0)
        for rr in range(_RPS // _ZR):
            pltpu.async_copy(zbuf, acc_sh.at[pl.ds(sid * _RPS + rr * _ZR, _ZR)],
                             zsem)
        for rr in range(_RPS // _ZR):
            pltpu.make_async_copy(
                zbuf, acc_sh.at[pl.ds(sid * _RPS + rr * _ZR, _ZR)], zsem).wait()
        pltpu.make_async_copy(row_hbm.at[wid], row_v, isem).wait()
        pltpu.make_async_copy(col_hbm.at[wid], col_v, isem).wait()
        pltpu.make_async_copy(val_hbm.at[wid], val_v, isem).wait()
        plsc.subcore_barrier()

        def scale(rb, t):
            def grp(g, c2):
                vv = val_v[t, pl.ds(g * _LANES, _LANES)]
                for k in range(_LANES):
                    v = vv[k]
                    e = g * _LANES + k
                    for j in range(jw):
                        sl = pl.ds(j * _LANES, _LANES)
                        rb[e, sl] = rb[e, sl] * v
                return c2

            lax.fori_loop(0, _CB // _LANES, grp, 0)

        pltpu.async_copy(y_hbm.at[col_v.at[0]], rb0, gs0)
        pltpu.async_copy(y_hbm.at[col_v.at[1]], rb1, gs1)

        def quad(q, carry):
            for i in range(4):
                t = 4 * q + i
                pltpu.make_async_copy(y_hbm.at[col_v.at[t]], rbs[i],
                                      gss[i]).wait()
                scale(rbs[i], t)
                pltpu.async_copy(rbs[i], acc_sh.at[row_v.at[t]], sss[i],
                                 add=True)
                i2 = (i + 2) % 4
                if i < 2:
                    @pl.when(q > 0)
                    def _():
                        pltpu.make_async_copy(
                            rbs[i2], acc_sh.at[row_v.at[t]], sss[i2]).wait()

                    pltpu.async_copy(y_hbm.at[col_v.at[t + 2]], rbs[i2],
                                     gss[i2])
                else:
                    pltpu.make_async_copy(
                        rbs[i2], acc_sh.at[row_v.at[t]], sss[i2]).wait()

                    @pl.when(q < _NQ - 1)
                    def _():
                        pltpu.async_copy(y_hbm.at[col_v.at[t + 2]], rbs[i2],
                                         gss[i2])

            return carry

        lax.fori_loop(0, _NQ, quad, 0)
        pltpu.make_async_copy(rb2, acc_sh.at[row_v.at[_TPC - 2]], ss2).wait()
        pltpu.make_async_copy(rb3, acc_sh.at[row_v.at[_TPC - 1]], ss3).wait()

        plsc.subcore_barrier()
        pltpu.sync_copy(acc_sh.at[pl.ds(sid * _RPS, _RPS)],
                        out_hbm.at[cid, pl.ds(sid * _RPS, _RPS)])

    return pl.kernel(
        body,
        out_type=jax.ShapeDtypeStruct((_NC, _N, width), jnp.float32),
        mesh=mesh,
        compiler_params=pltpu.CompilerParams(use_tc_tiling_on_sc=False),
        scratch_types=[
            pltpu.VMEM_SHARED((_N, width), jnp.float32),
            pltpu.VMEM((_CB, width), jnp.float32),
            pltpu.VMEM((_CB, width), jnp.float32),
            pltpu.VMEM((_CB, width), jnp.float32),
            pltpu.VMEM((_CB, width), jnp.float32),
            pltpu.VMEM((_TPC, _CB), jnp.int32),
            pltpu.VMEM((_TPC, _CB), jnp.int32),
            pltpu.VMEM((_TPC, _CB), jnp.float32),
            pltpu.VMEM((_ZR, width), jnp.float32),
            pltpu.SemaphoreType.DMA,
            pltpu.SemaphoreType.DMA,
            pltpu.SemaphoreType.DMA,
            pltpu.SemaphoreType.DMA,
            pltpu.SemaphoreType.DMA,
            pltpu.SemaphoreType.DMA,
            pltpu.SemaphoreType.DMA,
            pltpu.SemaphoreType.DMA,
            pltpu.SemaphoreType.DMA,
            pltpu.SemaphoreType.DMA,
        ],
    )


def _spmm(y, idx, val):
    cb = 32 if y.shape[1] > 128 else 128
    tpc = _EPW // cb
    row = idx[0].reshape(_NW, tpc, cb)
    col = idx[1].reshape(_NW, tpc, cb)
    v3 = val.reshape(_NW, tpc, cb)
    return _make_spmm(y.shape[1])(y, row, col, v3)


_EPW2 = _E // _NS


@functools.lru_cache(maxsize=None)
def _make_spmm2(width):
    mesh = plsc.VectorSubcoreMesh(core_axis_name="c", subcore_axis_name="s")
    jw = width // _LANES
    _CB = 64 if width >= 128 else 128
    _TPC = _EPW2 // _CB
    _NQ = _TPC // 4

    def body(y_hbm, row_hbm, col_hbm, val_hbm, out_hbm,
             acc_sh, rb0, rb1, rb2, rb3, row_v, col_v, val_v, zbuf,
             gs0, gs1, gs2, gs3, ss0, ss1, ss2, ss3, isem, zsem):
        cid = lax.axis_index("c")
        sid = lax.axis_index("s")
        rbs = (rb0, rb1, rb2, rb3)
        gss = (gs0, gs1, gs2, gs3)
        sss = (ss0, ss1, ss2, ss3)
        ysrc = y_hbm.at[cid]

        pltpu.async_copy(row_hbm.at[cid, sid], row_v, isem)
        pltpu.async_copy(col_hbm.at[cid, sid], col_v, isem)
        pltpu.async_copy(val_hbm.at[cid, sid], val_v, isem)

        zero16 = jnp.zeros((_LANES,), jnp.float32)

        def zrow(r, carry):
            for j in range(jw):
                zbuf[r, pl.ds(j * _LANES, _LANES)] = zero16
            return carry

        lax.fori_loop(0, _ZR // 2, zrow, 0)
        nzc = _RPS // (_ZR // 2)
        for rr in range(nzc):
            pltpu.async_copy(
                zbuf, acc_sh.at[pl.ds(sid * _RPS + rr * (_ZR // 2), _ZR // 2)],
                zsem)
        for rr in range(nzc):
            pltpu.make_async_copy(
                zbuf, acc_sh.at[pl.ds(sid * _RPS + rr * (_ZR // 2), _ZR // 2)],
                zsem).wait()
        pltpu.make_async_copy(row_hbm.at[cid, sid], row_v, isem).wait()
        pltpu.make_async_copy(col_hbm.at[cid, sid], col_v, isem).wait()
        pltpu.make_async_copy(val_hbm.at[cid, sid], val_v, isem).wait()
        plsc.subcore_barrier()

        def scale(rb, t):
            def grp(g, c2):
                vv = val_v[t, pl.ds(g * _LANES, _LANES)]
                for k in range(_LANES):
                    v = vv[k]
                    e = g * _LANES + k
                    for j in range(jw):
                        sl = pl.ds(j * _LANES, _LANES)
                        rb[e, sl] = rb[e, sl] * v
                return c2

            lax.fori_loop(0, _CB // _LANES, grp, 0)

        pltpu.async_copy(ysrc.at[col_v.at[0]], rb0, gs0)
        pltpu.async_copy(ysrc.at[col_v.at[1]], rb1, gs1)

        def quad(q, carry):
            for i in range(4):
                t = 4 * q + i
                pltpu.make_async_copy(ysrc.at[col_v.at[t]], rbs[i],
                                      gss[i]).wait()
                scale(rbs[i], t)
                pltpu.async_copy(rbs[i], acc_sh.at[row_v.at[t]], sss[i],
                                 add=True)
                i2 = (i + 2) % 4
                if i < 2:
                    @pl.when(q > 0)
                    def _():
                        pltpu.make_async_copy(
                            rbs[i2], acc_sh.at[row_v.at[t]], sss[i2]).wait()

                    pltpu.async_copy(ysrc.at[col_v.at[t + 2]], rbs[i2],
                                     gss[i2])
                else:
                    pltpu.make_async_copy(
                        rbs[i2], acc_sh.at[row_v.at[t]], sss[i2]).wait()

                    @pl.when(q < _NQ - 1)
                    def _():
                        pltpu.async_copy(ysrc.at[col_v.at[t + 2]], rbs[i2],
                                         gss[i2])

            return carry

        lax.fori_loop(0, _NQ, quad, 0)
        pltpu.make_async_copy(rb2, acc_sh.at[row_v.at[_TPC - 2]], ss2).wait()
        pltpu.make_async_copy(rb3, acc_sh.at[row_v.at[_TPC - 1]], ss3).wait()

        plsc.subcore_barrier()
        pltpu.sync_copy(acc_sh.at[pl.ds(sid * _RPS, _RPS)],
                        out_hbm.at[cid, pl.ds(sid * _RPS, _RPS)])

    return pl.kernel(
        body,
        out_type=jax.ShapeDtypeStruct((2, _N, width), jnp.float32),
        mesh=mesh,
        compiler_params=pltpu.CompilerParams(use_tc_tiling_on_sc=False),
        scratch_types=[
            pltpu.VMEM_SHARED((_N, width), jnp.float32),
            pltpu.VMEM((_CB, width), jnp.float32),
            pltpu.VMEM((_CB, width), jnp.float32),
            pltpu.VMEM((_CB, width), jnp.float32),
            pltpu.VMEM((_CB, width), jnp.float32),
            pltpu.VMEM((_TPC, _CB), jnp.int32),
            pltpu.VMEM((_TPC, _CB), jnp.int32),
            pltpu.VMEM((_TPC, _CB), jnp.float32),
            pltpu.VMEM((_ZR // 2, width), jnp.float32),
            pltpu.SemaphoreType.DMA,
            pltpu.SemaphoreType.DMA,
            pltpu.SemaphoreType.DMA,
            pltpu.SemaphoreType.DMA,
            pltpu.SemaphoreType.DMA,
            pltpu.SemaphoreType.DMA,
            pltpu.SemaphoreType.DMA,
            pltpu.SemaphoreType.DMA,
            pltpu.SemaphoreType.DMA,
            pltpu.SemaphoreType.DMA,
        ],
    )


@functools.lru_cache(maxsize=None)
def _make_spmm2x(width):
    mesh = plsc.VectorSubcoreMesh(core_axis_name="c", subcore_axis_name="s")
    jw = width // _LANES
    _CB = 64 if width >= 128 else 128
    _TPC = _EPW2 // _CB
    _NQ = _TPC // 4

    def body(y_hbm, row_hbm, col_hbm, val_hbm, outa_hbm, outb_hbm,
             acca_sh, accb_sh, rb0, rb1, rb2, rb3, row_v, col_v, val_v, zbuf,
             gs0, gs1, gs2, gs3, ss0, ss1, ss2, ss3, isem, zsem, esem):
        cid = lax.axis_index("c")
        sid = lax.axis_index("s")
        rbs = (rb0, rb1, rb2, rb3)
        gss = (gs0, gs1, gs2, gs3)
        sss = (ss0, ss1, ss2, ss3)

        pltpu.async_copy(row_hbm.at[cid, sid], row_v, isem)
        pltpu.async_copy(col_hbm.at[cid, sid], col_v, isem)
        pltpu.async_copy(val_hbm.at[cid, sid], val_v, isem)

        zero16 = jnp.zeros((_LANES,), jnp.float32)

        def zrow(r, carry):
            for j in range(jw):
                zbuf[r, pl.ds(j * _LANES, _LANES)] = zero16
            return carry

        lax.fori_loop(0, _ZR // 2, zrow, 0)
        nzc = _RPS // (_ZR // 2)
        for acc in (acca_sh, accb_sh):
            for rr in range(nzc):
                pltpu.async_copy(
                    zbuf,
                    acc.at[pl.ds(sid * _RPS + rr * (_ZR // 2), _ZR // 2)],
                    zsem)
        for acc in (acca_sh, accb_sh):
            for rr in range(nzc):
                pltpu.make_async_copy(
                    zbuf,
                    acc.at[pl.ds(sid * _RPS + rr * (_ZR // 2), _ZR // 2)],
                    zsem).wait()
        pltpu.make_async_copy(row_hbm.at[cid, sid], row_v, isem).wait()
        pltpu.make_async_copy(col_hbm.at[cid, sid], col_v, isem).wait()
        pltpu.make_async_copy(val_hbm.at[cid, sid], val_v, isem).wait()
        plsc.subcore_barrier()

        def scale(rb, t):
            def grp(g, c2):
                vv = val_v[t, pl.ds(g * _LANES, _LANES)]
                for k in range(_LANES):
                    v = vv[k]
                    e = g * _LANES + k
                    for j in range(jw):
                        sl = pl.ds(j * _LANES, _LANES)
                        rb[e, sl] = rb[e, sl] * v
                return c2

            lax.fori_loop(0, _CB // _LANES, grp, 0)

        def hop(ysrc, acc):
            pltpu.async_copy(ysrc.at[col_v.at[0]], rb0, gs0)
            pltpu.async_copy(ysrc.at[col_v.at[1]], rb1, gs1)

            def quad(q, carry):
                for i in range(4):
                    t = 4 * q + i
                    pltpu.make_async_copy(ysrc.at[col_v.at[t]], rbs[i],
                                          gss[i]).wait()
                    scale(rbs[i], t)
                    pltpu.async_copy(rbs[i], acc.at[row_v.at[t]], sss[i],
                                     add=True)
                    i2 = (i + 2) % 4
                    if i < 2:
                        @pl.when(q > 0)
                        def _():
                            pltpu.make_async_copy(
                                rbs[i2], acc.at[row_v.at[t]], sss[i2]).wait()

                        pltpu.async_copy(ysrc.at[col_v.at[t + 2]], rbs[i2],
                                         gss[i2])
                    else:
                        pltpu.make_async_copy(
                            rbs[i2], acc.at[row_v.at[t]], sss[i2]).wait()

                        @pl.when(q < _NQ - 1)
                        def _():
                            pltpu.async_copy(ysrc.at[col_v.at[t + 2]],
                                             rbs[i2], gss[i2])

                return carry

            lax.fori_loop(0, _NQ, quad, 0)
            pltpu.make_async_copy(rb2, acc.at[row_v.at[_TPC - 2]], ss2).wait()
            pltpu.make_async_copy(rb3, acc.at[row_v.at[_TPC - 1]], ss3).wait()

        hop(y_hbm.at[cid], acca_sh)
        plsc.subcore_barrier()
        pltpu.async_copy(acca_sh.at[pl.ds(sid * _RPS, _RPS)],
                         outa_hbm.at[cid, pl.ds(sid * _RPS, _RPS)], esem)
        hop(acca_sh, accb_sh)
        pltpu.make_async_copy(acca_sh.at[pl.ds(sid * _RPS, _RPS)],
                              outa_hbm.at[cid, pl.ds(sid * _RPS, _RPS)],
                              esem).wait()
        plsc.subcore_barrier()
        pltpu.sync_copy(accb_sh.at[pl.ds(sid * _RPS, _RPS)],
                        outb_hbm.at[cid, pl.ds(sid * _RPS, _RPS)])

    return pl.kernel(
        body,
        out_type=[jax.ShapeDtypeStruct((2, _N, width), jnp.float32),
                  jax.ShapeDtypeStruct((2, _N, width), jnp.float32)],
        mesh=mesh,
        compiler_params=pltpu.CompilerParams(use_tc_tiling_on_sc=False),
        scratch_types=[
            pltpu.VMEM_SHARED((_N, width), jnp.float32),
            pltpu.VMEM_SHARED((_N, width), jnp.float32),
            pltpu.VMEM((_CB, width), jnp.float32),
            pltpu.VMEM((_CB, width), jnp.float32),
            pltpu.VMEM((_CB, width), jnp.float32),
            pltpu.VMEM((_CB, width), jnp.float32),
            pltpu.VMEM((_TPC, _CB), jnp.int32),
            pltpu.VMEM((_TPC, _CB), jnp.int32),
            pltpu.VMEM((_TPC, _CB), jnp.float32),
            pltpu.VMEM((_ZR // 2, width), jnp.float32),
            pltpu.SemaphoreType.DMA,
            pltpu.SemaphoreType.DMA,
            pltpu.SemaphoreType.DMA,
            pltpu.SemaphoreType.DMA,
            pltpu.SemaphoreType.DMA,
            pltpu.SemaphoreType.DMA,
            pltpu.SemaphoreType.DMA,
            pltpu.SemaphoreType.DMA,
            pltpu.SemaphoreType.DMA,
            pltpu.SemaphoreType.DMA,
            pltpu.SemaphoreType.DMA,
        ],
    )


def _spmm2x(y_s, edges):
    row, col, v = edges
    return _make_spmm2x(y_s.shape[2])(y_s, row, col, v)


def _edges2(idx1, val1, idx2, val2, width):
    cb = 64 if width >= 128 else 128
    tpc = _EPW2 // cb
    row = jnp.stack([idx1[0].reshape(_NS, tpc, cb),
                     idx2[0].reshape(_NS, tpc, cb)])
    col = jnp.stack([idx1[1].reshape(_NS, tpc, cb),
                     idx2[1].reshape(_NS, tpc, cb)])
    v = jnp.stack([val1.reshape(_NS, tpc, cb), val2.reshape(_NS, tpc, cb)])
    return row, col, v


def _spmm2(y_s, edges):
    row, col, v = edges
    return _make_spmm2(y_s.shape[2])(y_s, row, col, v)


_BN = 1024


def _dot_t(a, b):
    return lax.dot_general(a, b, (((1,), (1,)), ((), ())),
                           preferred_element_type=jnp.float32)


def _lrelu(x):
    return jnp.where(x >= 0, x, 0.2 * x)


def _mm_body(x_ref, w_ref, o_ref):
    o_ref[...] = jnp.dot(x_ref[...], w_ref[...],
                         preferred_element_type=jnp.float32)


def _mm(x, w):
    n, din = x.shape
    dout = w.shape[1]
    return pl.pallas_call(
        _mm_body,
        grid=(n // _BN,),
        in_specs=[pl.BlockSpec((_BN, din), lambda i: (i, 0)),
                  pl.BlockSpec((din, dout), lambda i: (0, 0))],
        out_specs=pl.BlockSpec((_BN, dout), lambda i: (i, 0)),
        out_shape=jax.ShapeDtypeStruct((n, dout), jnp.float32),
    )(x, w)


def _fuse_body(act, p_ref, w_ref, o_ref):
    a = p_ref[0] + p_ref[1]
    if act:
        a = _lrelu(a)
    o_ref[...] = jnp.dot(a, w_ref[...], preferred_element_type=jnp.float32)


def _fuse(p, w, act=True):
    _, n, din = p.shape
    dout = w.shape[1]
    return pl.pallas_call(
        functools.partial(_fuse_body, act),
        grid=(n // _BN,),
        in_specs=[pl.BlockSpec((2, _BN, din), lambda i: (0, i, 0)),
                  pl.BlockSpec((din, dout), lambda i: (0, 0))],
        out_specs=pl.BlockSpec((_BN, dout), lambda i: (i, 0)),
        out_shape=jax.ShapeDtypeStruct((n, dout), jnp.float32),
    )(p, w)


def _fuse2_body(act, s_ref, w_ref, o_ref):
    a = s_ref[0]
    if act:
        a = _lrelu(a)
    o_ref[0] = jnp.dot(a, w_ref[0], preferred_element_type=jnp.float32)


def _fuse2(s, w_s, act=True):
    _, n, din = s.shape
    dout = w_s.shape[2]
    return pl.pallas_call(
        functools.partial(_fuse2_body, act),
        grid=(2, n // _BN),
        in_specs=[pl.BlockSpec((1, _BN, din), lambda v, i: (v, i, 0)),
                  pl.BlockSpec((1, din, dout), lambda v, i: (v, 0, 0))],
        out_specs=pl.BlockSpec((1, _BN, dout), lambda v, i: (v, i, 0)),
        out_shape=jax.ShapeDtypeStruct((2, n, dout), jnp.float32),
    )(s, w_s)


def _fuse1_body(act, s_ref, w_ref, o_ref):
    a = s_ref[0]
    if act:
        a = _lrelu(a)
    o_ref[...] = jnp.dot(a, w_ref[...], preferred_element_type=jnp.float32)


def _fuse1(s, view, w, act=True):
    _, n, din = s.shape
    dout = w.shape[1]
    return pl.pallas_call(
        functools.partial(_fuse1_body, act),
        grid=(n // _BN,),
        in_specs=[pl.BlockSpec((1, _BN, din), lambda i: (view, i, 0)),
                  pl.BlockSpec((din, dout), lambda i: (0, 0))],
        out_specs=pl.BlockSpec((_BN, dout), lambda i: (i, 0)),
        out_shape=jax.ShapeDtypeStruct((n, dout), jnp.float32),
    )(s, w)


def _add_body(p_ref, o_ref):
    o_ref[...] = p_ref[0] + p_ref[1]


def _add(p):
    _, n, d = p.shape
    return pl.pallas_call(
        _add_body,
        grid=(n // _BN,),
        in_specs=[pl.BlockSpec((2, _BN, d), lambda i: (0, i, 0))],
        out_specs=pl.BlockSpec((_BN, d), lambda i: (i, 0)),
        out_shape=jax.ShapeDtypeStruct((n, d), jnp.float32),
    )(p)


def _addmm_body(p_ref, w_ref, z_ref, t_ref):
    z = p_ref[0] + p_ref[1]
    z_ref[...] = z
    t_ref[...] = jnp.dot(z, w_ref[...], preferred_element_type=jnp.float32)


def _addmm(p, w):
    _, n, din = p.shape
    dout = w.shape[1]
    return pl.pallas_call(
        _addmm_body,
        grid=(n // _BN,),
        in_specs=[pl.BlockSpec((2, _BN, din), lambda i: (0, i, 0)),
                  pl.BlockSpec((din, dout), lambda i: (0, 0))],
        out_specs=[pl.BlockSpec((_BN, din), lambda i: (i, 0)),
                   pl.BlockSpec((_BN, dout), lambda i: (i, 0))],
        out_shape=[jax.ShapeDtypeStruct((n, din), jnp.float32),
                   jax.ShapeDtypeStruct((n, dout), jnp.float32)],
    )(p, w)


_BADJ = 1024


def _adj_body(zi_i, zh_i, zi_j, zh_j, o_ref):
    g1 = _dot_t(zi_i[...], zi_j[...])
    g2 = _dot_t(zh_i[...], zh_j[...])
    o_ref[...] = jax.nn.sigmoid(g1) + jax.nn.sigmoid(g2)


def _adj(zi, zh):
    n, dz = zi.shape
    dh = zh.shape[1]
    return pl.pallas_call(
        _adj_body,
        grid=(n // _BADJ, n // _BADJ),
        in_specs=[pl.BlockSpec((_BADJ, dz), lambda i, j: (i, 0)),
                  pl.BlockSpec((_BADJ, dh), lambda i, j: (i, 0)),
                  pl.BlockSpec((_BADJ, dz), lambda i, j: (j, 0)),
                  pl.BlockSpec((_BADJ, dh), lambda i, j: (j, 0))],
        out_specs=pl.BlockSpec((_BADJ, _BADJ), lambda i, j: (i, j)),
        out_shape=jax.ShapeDtypeStruct((n, n), jnp.float32),
    )(zi, zh, zi, zh)


def _st_body(z_ref, c_ref, o_ref):
    z = z_ref[0]
    c = c_ref[0]
    zz = jnp.sum(z * z, axis=1, keepdims=True)
    cc = jnp.sum(c * c, axis=1).reshape(1, -1)
    d = zz - 2.0 * _dot_t(z, c) + cc
    q = 1.0 / (1.0 + d)
    o_ref[0] = q / jnp.sum(q, axis=1, keepdims=True)


def _student_t4(z4, c2):
    _, n, dz = z4.shape
    ncl = c2.shape[1]
    return pl.pallas_call(
        _st_body,
        grid=(4,),
        in_specs=[pl.BlockSpec((1, n, dz), lambda v: (v, 0, 0)),
                  pl.BlockSpec((1, ncl, dz), lambda v: (v // 2, 0, 0))],
        out_specs=pl.BlockSpec((1, n, ncl), lambda v: (v, 0, 0)),
        out_shape=jax.ShapeDtypeStruct((4, n, ncl), jnp.float32),
    )(z4, c2)


def kernel(x1, adj1_idx, adj1_val, x2, adj2_idx, adj2_val,
           w_e1_1, w_e1_2, w_e1_3, w_d1_1, w_d1_2, w_d1_3,
           w_e2_1, w_e2_2, w_e2_3, w_d2_1, w_d2_2, w_d2_3,
           centers1, centers2):
    ewide = _edges2(adj1_idx, adj1_val, adj2_idx, adj2_val, 128)
    enarrow = _edges2(adj1_idx, adj1_val, adj2_idx, adj2_val, 64)

    ys = jnp.stack([_mm(x1, w_e1_1), _mm(x2, w_e2_1)])
    s = _spmm2(ys, ewide)
    ys = _fuse2(s, jnp.stack([w_e1_2, w_e2_2]))
    s = _spmm2(ys, enarrow)
    ys = _fuse2(s, jnp.stack([w_e1_3, w_e2_3]))
    zi_s, z1_s = _spmm2x(ys, enarrow)
    ys = _fuse2(z1_s, jnp.stack([w_d1_1, w_d2_1]), act=False)
    s = _spmm2(ys, enarrow)
    ys = _fuse2(s, jnp.stack([w_d1_2, w_d2_2]))
    s = _spmm2(ys, ewide)
    t1 = _fuse1(s, 0, w_d1_3)
    t2 = _fuse1(s, 1, w_d2_3)
    zi1, zi2 = zi_s[0], zi_s[1]
    z1, z2 = z1_s[0], z1_s[1]
    z_hat1 = _add(_spmm(t1, adj1_idx, adj1_val))
    a_hat1 = _adj(zi1, z_hat1)
    z_hat2 = _add(_spmm(t2, adj2_idx, adj2_val))
    a_hat2 = _adj(zi2, z_hat2)
    q4 = _student_t4(jnp.stack([z1, zi1, z2, zi2]),
                     jnp.stack([centers1, centers2]))
    Q1 = (q4[0], q4[1])
    Q2 = (q4[2], q4[3])
    return (z_hat1, a_hat1, z_hat2, a_hat2, Q1, Q2, z1, z2, (z1, z2))


# speedup vs baseline: 7.0592x; 1.1188x over previous
import functools

import jax
import jax.numpy as jnp
from jax import lax
from jax.experimental import pallas as pl
from jax.experimental.pallas import tpu as pltpu
from jax.experimental.pallas import tpu_sc as plsc

_N = 4096
_E = 65536
_NZ = 32
_NCL = 10

_NC = 2
_NS = 16
_LANES = 16
_NW = _NC * _NS
_EPW = _E // _NW
_RPS = _N // _NS
_ZR = 64


@functools.lru_cache(maxsize=None)
def _make_spmm(width):
    mesh = plsc.VectorSubcoreMesh(core_axis_name="c", subcore_axis_name="s")
    jw = width // _LANES
    _CB = 32 if width > 128 else 128
    _TPC = _EPW // _CB
    _NQ = _TPC // 4

    def body(y_hbm, row_hbm, col_hbm, val_hbm, out_hbm,
             acc_sh, rb0, rb1, rb2, rb3, row_v, col_v, val_v, zbuf,
             gs0, gs1, gs2, gs3, ss0, ss1, ss2, ss3, isem, zsem):
        cid = lax.axis_index("c")
        sid = lax.axis_index("s")
        wid = sid * _NC + cid
        rbs = (rb0, rb1, rb2, rb3)
        gss = (gs0, gs1, gs2, gs3)
        sss = (ss0, ss1, ss2, ss3)

        pltpu.async_copy(row_hbm.at[wid], row_v, isem)
        pltpu.async_copy(col_hbm.at[wid], col_v, isem)
        pltpu.async_copy(val_hbm.at[wid], val_v, isem)

        zero16 = jnp.zeros((_LANES,), jnp.float32)

        def zrow(r, carry):
            for j in range(jw):
                zbuf[r, pl.ds(j * _LANES, _LANES)] = zero16
            return carry

        lax.fori_loop(0, _ZR, zrow, 0)
        for rr in range(_RPS // _ZR):
            pltpu.async_copy(zbuf, acc_sh.at[pl.ds(sid * _RPS + rr * _ZR, _ZR)],
                             zsem)
        for rr in range(_RPS // _ZR):
            pltpu.make_async_copy(
                zbuf, acc_sh.at[pl.ds(sid * _RPS + rr * _ZR, _ZR)], zsem).wait()
        pltpu.make_async_copy(row_hbm.at[wid], row_v, isem).wait()
        pltpu.make_async_copy(col_hbm.at[wid], col_v, isem).wait()
        pltpu.make_async_copy(val_hbm.at[wid], val_v, isem).wait()
        plsc.subcore_barrier()

        def scale(rb, t):
            @plsc.parallel_loop(0, _CB // _LANES, unroll=2)
            def grp(g):
                vv = val_v[t, pl.ds(g * _LANES, _LANES)]
                for k in range(_LANES):
                    v = vv[k]
                    e = g * _LANES + k
                    for j in range(jw):
                        sl = pl.ds(j * _LANES, _LANES)
                        rb[e, sl] = rb[e, sl] * v

        pltpu.async_copy(y_hbm.at[col_v.at[0]], rb0, gs0)
        pltpu.async_copy(y_hbm.at[col_v.at[1]], rb1, gs1)

        def quad(q, carry):
            for i in range(4):
                t = 4 * q + i
                pltpu.make_async_copy(y_hbm.at[col_v.at[t]], rbs[i],
                                      gss[i]).wait()
                scale(rbs[i], t)
                pltpu.async_copy(rbs[i], acc_sh.at[row_v.at[t]], sss[i],
                                 add=True)
                i2 = (i + 2) % 4
                if i < 2:
                    @pl.when(q > 0)
                    def _():
                        pltpu.make_async_copy(
                            rbs[i2], acc_sh.at[row_v.at[t]], sss[i2]).wait()

                    pltpu.async_copy(y_hbm.at[col_v.at[t + 2]], rbs[i2],
                                     gss[i2])
                else:
                    pltpu.make_async_copy(
                        rbs[i2], acc_sh.at[row_v.at[t]], sss[i2]).wait()

                    @pl.when(q < _NQ - 1)
                    def _():
                        pltpu.async_copy(y_hbm.at[col_v.at[t + 2]], rbs[i2],
                                         gss[i2])

            return carry

        lax.fori_loop(0, _NQ, quad, 0)
        pltpu.make_async_copy(rb2, acc_sh.at[row_v.at[_TPC - 2]], ss2).wait()
        pltpu.make_async_copy(rb3, acc_sh.at[row_v.at[_TPC - 1]], ss3).wait()

        plsc.subcore_barrier()
        pltpu.sync_copy(acc_sh.at[pl.ds(sid * _RPS, _RPS)],
                        out_hbm.at[cid, pl.ds(sid * _RPS, _RPS)])

    return pl.kernel(
        body,
        out_type=jax.ShapeDtypeStruct((_NC, _N, width), jnp.float32),
        mesh=mesh,
        compiler_params=pltpu.CompilerParams(use_tc_tiling_on_sc=False),
        scratch_types=[
            pltpu.VMEM_SHARED((_N, width), jnp.float32),
            pltpu.VMEM((_CB, width), jnp.float32),
            pltpu.VMEM((_CB, width), jnp.float32),
            pltpu.VMEM((_CB, width), jnp.float32),
            pltpu.VMEM((_CB, width), jnp.float32),
            pltpu.VMEM((_TPC, _CB), jnp.int32),
            pltpu.VMEM((_TPC, _CB), jnp.int32),
            pltpu.VMEM((_TPC, _CB), jnp.float32),
            pltpu.VMEM((_ZR, width), jnp.float32),
            pltpu.SemaphoreType.DMA,
            pltpu.SemaphoreType.DMA,
            pltpu.SemaphoreType.DMA,
            pltpu.SemaphoreType.DMA,
            pltpu.SemaphoreType.DMA,
            pltpu.SemaphoreType.DMA,
            pltpu.SemaphoreType.DMA,
            pltpu.SemaphoreType.DMA,
            pltpu.SemaphoreType.DMA,
            pltpu.SemaphoreType.DMA,
        ],
    )


def _spmm(y, idx, val):
    cb = 32 if y.shape[1] > 128 else 128
    tpc = _EPW // cb
    row = idx[0].reshape(_NW, tpc, cb)
    col = idx[1].reshape(_NW, tpc, cb)
    v3 = val.reshape(_NW, tpc, cb)
    return _make_spmm(y.shape[1])(y, row, col, v3)


_EPW2 = _E // _NS


@functools.lru_cache(maxsize=None)
def _make_spmm2(width):
    mesh = plsc.VectorSubcoreMesh(core_axis_name="c", subcore_axis_name="s")
    jw = width // _LANES
    _CB = 64 if width >= 128 else 128
    _TPC = _EPW2 // _CB
    _NQ = _TPC // 4

    def body(y_hbm, row_hbm, col_hbm, val_hbm, out_hbm,
             acc_sh, rb0, rb1, rb2, rb3, row_v, col_v, val_v, zbuf,
             gs0, gs1, gs2, gs3, ss0, ss1, ss2, ss3, isem, zsem):
        cid = lax.axis_index("c")
        sid = lax.axis_index("s")
        rbs = (rb0, rb1, rb2, rb3)
        gss = (gs0, gs1, gs2, gs3)
        sss = (ss0, ss1, ss2, ss3)
        ysrc = y_hbm.at[cid]

        pltpu.async_copy(row_hbm.at[cid, sid], row_v, isem)
        pltpu.async_copy(col_hbm.at[cid, sid], col_v, isem)
        pltpu.async_copy(val_hbm.at[cid, sid], val_v, isem)

        zero16 = jnp.zeros((_LANES,), jnp.float32)

        def zrow(r, carry):
            for j in range(jw):
                zbuf[r, pl.ds(j * _LANES, _LANES)] = zero16
            return carry

        lax.fori_loop(0, _ZR // 2, zrow, 0)
        nzc = _RPS // (_ZR // 2)
        for rr in range(nzc):
            pltpu.async_copy(
                zbuf, acc_sh.at[pl.ds(sid * _RPS + rr * (_ZR // 2), _ZR // 2)],
                zsem)
        for rr in range(nzc):
            pltpu.make_async_copy(
                zbuf, acc_sh.at[pl.ds(sid * _RPS + rr * (_ZR // 2), _ZR // 2)],
                zsem).wait()
        pltpu.make_async_copy(row_hbm.at[cid, sid], row_v, isem).wait()
        pltpu.make_async_copy(col_hbm.at[cid, sid], col_v, isem).wait()
        pltpu.make_async_copy(val_hbm.at[cid, sid], val_v, isem).wait()
        plsc.subcore_barrier()

        def scale(rb, t):
            @plsc.parallel_loop(0, _CB // _LANES, unroll=2)
            def grp(g):
                vv = val_v[t, pl.ds(g * _LANES, _LANES)]
                for k in range(_LANES):
                    v = vv[k]
                    e = g * _LANES + k
                    for j in range(jw):
                        sl = pl.ds(j * _LANES, _LANES)
                        rb[e, sl] = rb[e, sl] * v

        pltpu.async_copy(ysrc.at[col_v.at[0]], rb0, gs0)
        pltpu.async_copy(ysrc.at[col_v.at[1]], rb1, gs1)

        def quad(q, carry):
            for i in range(4):
                t = 4 * q + i
                pltpu.make_async_copy(ysrc.at[col_v.at[t]], rbs[i],
                                      gss[i]).wait()
                scale(rbs[i], t)
                pltpu.async_copy(rbs[i], acc_sh.at[row_v.at[t]], sss[i],
                                 add=True)
                i2 = (i + 2) % 4
                if i < 2:
                    @pl.when(q > 0)
                    def _():
                        pltpu.make_async_copy(
                            rbs[i2], acc_sh.at[row_v.at[t]], sss[i2]).wait()

                    pltpu.async_copy(ysrc.at[col_v.at[t + 2]], rbs[i2],
                                     gss[i2])
                else:
                    pltpu.make_async_copy(
                        rbs[i2], acc_sh.at[row_v.at[t]], sss[i2]).wait()

                    @pl.when(q < _NQ - 1)
                    def _():
                        pltpu.async_copy(ysrc.at[col_v.at[t + 2]], rbs[i2],
                                         gss[i2])

            return carry

        lax.fori_loop(0, _NQ, quad, 0)
        pltpu.make_async_copy(rb2, acc_sh.at[row_v.at[_TPC - 2]], ss2).wait()
        pltpu.make_async_copy(rb3, acc_sh.at[row_v.at[_TPC - 1]], ss3).wait()

        plsc.subcore_barrier()
        pltpu.sync_copy(acc_sh.at[pl.ds(sid * _RPS, _RPS)],
                        out_hbm.at[cid, pl.ds(sid * _RPS, _RPS)])

    return pl.kernel(
        body,
        out_type=jax.ShapeDtypeStruct((2, _N, width), jnp.float32),
        mesh=mesh,
        compiler_params=pltpu.CompilerParams(use_tc_tiling_on_sc=False),
        scratch_types=[
            pltpu.VMEM_SHARED((_N, width), jnp.float32),
            pltpu.VMEM((_CB, width), jnp.float32),
            pltpu.VMEM((_CB, width), jnp.float32),
            pltpu.VMEM((_CB, width), jnp.float32),
            pltpu.VMEM((_CB, width), jnp.float32),
            pltpu.VMEM((_TPC, _CB), jnp.int32),
            pltpu.VMEM((_TPC, _CB), jnp.int32),
            pltpu.VMEM((_TPC, _CB), jnp.float32),
            pltpu.VMEM((_ZR // 2, width), jnp.float32),
            pltpu.SemaphoreType.DMA,
            pltpu.SemaphoreType.DMA,
            pltpu.SemaphoreType.DMA,
            pltpu.SemaphoreType.DMA,
            pltpu.SemaphoreType.DMA,
            pltpu.SemaphoreType.DMA,
            pltpu.SemaphoreType.DMA,
            pltpu.SemaphoreType.DMA,
            pltpu.SemaphoreType.DMA,
            pltpu.SemaphoreType.DMA,
        ],
    )


@functools.lru_cache(maxsize=None)
def _make_spmm2x(width):
    mesh = plsc.VectorSubcoreMesh(core_axis_name="c", subcore_axis_name="s")
    jw = width // _LANES
    _CB = 64 if width >= 128 else 128
    _TPC = _EPW2 // _CB
    _NQ = _TPC // 4

    def body(y_hbm, row_hbm, col_hbm, val_hbm, outa_hbm, outb_hbm,
             acca_sh, accb_sh, rb0, rb1, rb2, rb3, row_v, col_v, val_v, zbuf,
             gs0, gs1, gs2, gs3, ss0, ss1, ss2, ss3, isem, zsem, esem):
        cid = lax.axis_index("c")
        sid = lax.axis_index("s")
        rbs = (rb0, rb1, rb2, rb3)
        gss = (gs0, gs1, gs2, gs3)
        sss = (ss0, ss1, ss2, ss3)

        pltpu.async_copy(row_hbm.at[cid, sid], row_v, isem)
        pltpu.async_copy(col_hbm.at[cid, sid], col_v, isem)
        pltpu.async_copy(val_hbm.at[cid, sid], val_v, isem)

        zero16 = jnp.zeros((_LANES,), jnp.float32)

        def zrow(r, carry):
            for j in range(jw):
                zbuf[r, pl.ds(j * _LANES, _LANES)] = zero16
            return carry

        lax.fori_loop(0, _ZR // 2, zrow, 0)
        nzc = _RPS // (_ZR // 2)
        for acc in (acca_sh, accb_sh):
            for rr in range(nzc):
                pltpu.async_copy(
                    zbuf,
                    acc.at[pl.ds(sid * _RPS + rr * (_ZR // 2), _ZR // 2)],
                    zsem)
        for acc in (acca_sh, accb_sh):
            for rr in range(nzc):
                pltpu.make_async_copy(
                    zbuf,
                    acc.at[pl.ds(sid * _RPS + rr * (_ZR // 2), _ZR // 2)],
                    zsem).wait()
        pltpu.make_async_copy(row_hbm.at[cid, sid], row_v, isem).wait()
        pltpu.make_async_copy(col_hbm.at[cid, sid], col_v, isem).wait()
        pltpu.make_async_copy(val_hbm.at[cid, sid], val_v, isem).wait()
        plsc.subcore_barrier()

        def scale(rb, t):
            @plsc.parallel_loop(0, _CB // _LANES, unroll=2)
            def grp(g):
                vv = val_v[t, pl.ds(g * _LANES, _LANES)]
                for k in range(_LANES):
                    v = vv[k]
                    e = g * _LANES + k
                    for j in range(jw):
                        sl = pl.ds(j * _LANES, _LANES)
                        rb[e, sl] = rb[e, sl] * v

        def hop(ysrc, acc):
            pltpu.async_copy(ysrc.at[col_v.at[0]], rb0, gs0)
            pltpu.async_copy(ysrc.at[col_v.at[1]], rb1, gs1)

            def quad(q, carry):
                for i in range(4):
                    t = 4 * q + i
                    pltpu.make_async_copy(ysrc.at[col_v.at[t]], rbs[i],
                                          gss[i]).wait()
                    scale(rbs[i], t)
                    pltpu.async_copy(rbs[i], acc.at[row_v.at[t]], sss[i],
                                     add=True)
                    i2 = (i + 2) % 4
                    if i < 2:
                        @pl.when(q > 0)
                        def _():
                            pltpu.make_async_copy(
                                rbs[i2], acc.at[row_v.at[t]], sss[i2]).wait()

                        pltpu.async_copy(ysrc.at[col_v.at[t + 2]], rbs[i2],
                                         gss[i2])
                    else:
                        pltpu.make_async_copy(
                            rbs[i2], acc.at[row_v.at[t]], sss[i2]).wait()

                        @pl.when(q < _NQ - 1)
                        def _():
                            pltpu.async_copy(ysrc.at[col_v.at[t + 2]],
                                             rbs[i2], gss[i2])

                return carry

            lax.fori_loop(0, _NQ, quad, 0)
            pltpu.make_async_copy(rb2, acc.at[row_v.at[_TPC - 2]], ss2).wait()
            pltpu.make_async_copy(rb3, acc.at[row_v.at[_TPC - 1]], ss3).wait()

        hop(y_hbm.at[cid], acca_sh)
        plsc.subcore_barrier()
        pltpu.async_copy(acca_sh.at[pl.ds(sid * _RPS, _RPS)],
                         outa_hbm.at[cid, pl.ds(sid * _RPS, _RPS)], esem)
        hop(acca_sh, accb_sh)
        pltpu.make_async_copy(acca_sh.at[pl.ds(sid * _RPS, _RPS)],
                              outa_hbm.at[cid, pl.ds(sid * _RPS, _RPS)],
                              esem).wait()
        plsc.subcore_barrier()
        pltpu.sync_copy(accb_sh.at[pl.ds(sid * _RPS, _RPS)],
                        outb_hbm.at[cid, pl.ds(sid * _RPS, _RPS)])

    return pl.kernel(
        body,
        out_type=[jax.ShapeDtypeStruct((2, _N, width), jnp.float32),
                  jax.ShapeDtypeStruct((2, _N, width), jnp.float32)],
        mesh=mesh,
        compiler_params=pltpu.CompilerParams(use_tc_tiling_on_sc=False),
        scratch_types=[
            pltpu.VMEM_SHARED((_N, width), jnp.float32),
            pltpu.VMEM_SHARED((_N, width), jnp.float32),
            pltpu.VMEM((_CB, width), jnp.float32),
            pltpu.VMEM((_CB, width), jnp.float32),
            pltpu.VMEM((_CB, width), jnp.float32),
            pltpu.VMEM((_CB, width), jnp.float32),
            pltpu.VMEM((_TPC, _CB), jnp.int32),
            pltpu.VMEM((_TPC, _CB), jnp.int32),
            pltpu.VMEM((_TPC, _CB), jnp.float32),
            pltpu.VMEM((_ZR // 2, width), jnp.float32),
            pltpu.SemaphoreType.DMA,
            pltpu.SemaphoreType.DMA,
            pltpu.SemaphoreType.DMA,
            pltpu.SemaphoreType.DMA,
            pltpu.SemaphoreType.DMA,
            pltpu.SemaphoreType.DMA,
            pltpu.SemaphoreType.DMA,
            pltpu.SemaphoreType.DMA,
            pltpu.SemaphoreType.DMA,
            pltpu.SemaphoreType.DMA,
            pltpu.SemaphoreType.DMA,
        ],
    )


def _spmm2x(y_s, edges):
    row, col, v = edges
    return _make_spmm2x(y_s.shape[2])(y_s, row, col, v)


def _edges2(idx1, val1, idx2, val2, width):
    cb = 64 if width >= 128 else 128
    tpc = _EPW2 // cb
    row = jnp.stack([idx1[0].reshape(_NS, tpc, cb),
                     idx2[0].reshape(_NS, tpc, cb)])
    col = jnp.stack([idx1[1].reshape(_NS, tpc, cb),
                     idx2[1].reshape(_NS, tpc, cb)])
    v = jnp.stack([val1.reshape(_NS, tpc, cb), val2.reshape(_NS, tpc, cb)])
    return row, col, v


def _spmm2(y_s, edges):
    row, col, v = edges
    return _make_spmm2(y_s.shape[2])(y_s, row, col, v)


_BN = 1024


def _dot_t(a, b):
    return lax.dot_general(a, b, (((1,), (1,)), ((), ())),
                           preferred_element_type=jnp.float32)


def _lrelu(x):
    return jnp.where(x >= 0, x, 0.2 * x)


def _mm_body(x_ref, w_ref, o_ref):
    o_ref[...] = jnp.dot(x_ref[...], w_ref[...],
                         preferred_element_type=jnp.float32)


def _mm(x, w):
    n, din = x.shape
    dout = w.shape[1]
    return pl.pallas_call(
        _mm_body,
        grid=(n // _BN,),
        in_specs=[pl.BlockSpec((_BN, din), lambda i: (i, 0)),
                  pl.BlockSpec((din, dout), lambda i: (0, 0))],
        out_specs=pl.BlockSpec((_BN, dout), lambda i: (i, 0)),
        out_shape=jax.ShapeDtypeStruct((n, dout), jnp.float32),
    )(x, w)


def _fuse_body(act, p_ref, w_ref, o_ref):
    a = p_ref[0] + p_ref[1]
    if act:
        a = _lrelu(a)
    o_ref[...] = jnp.dot(a, w_ref[...], preferred_element_type=jnp.float32)


def _fuse(p, w, act=True):
    _, n, din = p.shape
    dout = w.shape[1]
    return pl.pallas_call(
        functools.partial(_fuse_body, act),
        grid=(n // _BN,),
        in_specs=[pl.BlockSpec((2, _BN, din), lambda i: (0, i, 0)),
                  pl.BlockSpec((din, dout), lambda i: (0, 0))],
        out_specs=pl.BlockSpec((_BN, dout), lambda i: (i, 0)),
        out_shape=jax.ShapeDtypeStruct((n, dout), jnp.float32),
    )(p, w)


def _fuse2_body(act, s_ref, w_ref, o_ref):
    a = s_ref[0]
    if act:
        a = _lrelu(a)
    o_ref[0] = jnp.dot(a, w_ref[0], preferred_element_type=jnp.float32)


def _fuse2(s, w_s, act=True):
    _, n, din = s.shape
    dout = w_s.shape[2]
    return pl.pallas_call(
        functools.partial(_fuse2_body, act),
        grid=(2, n // _BN),
        in_specs=[pl.BlockSpec((1, _BN, din), lambda v, i: (v, i, 0)),
                  pl.BlockSpec((1, din, dout), lambda v, i: (v, 0, 0))],
        out_specs=pl.BlockSpec((1, _BN, dout), lambda v, i: (v, i, 0)),
        out_shape=jax.ShapeDtypeStruct((2, n, dout), jnp.float32),
    )(s, w_s)


def _fuse1_body(act, s_ref, w_ref, o_ref):
    a = s_ref[0]
    if act:
        a = _lrelu(a)
    o_ref[...] = jnp.dot(a, w_ref[...], preferred_element_type=jnp.float32)


def _fuse1(s, view, w, act=True):
    _, n, din = s.shape
    dout = w.shape[1]
    return pl.pallas_call(
        functools.partial(_fuse1_body, act),
        grid=(n // _BN,),
        in_specs=[pl.BlockSpec((1, _BN, din), lambda i: (view, i, 0)),
                  pl.BlockSpec((din, dout), lambda i: (0, 0))],
        out_specs=pl.BlockSpec((_BN, dout), lambda i: (i, 0)),
        out_shape=jax.ShapeDtypeStruct((n, dout), jnp.float32),
    )(s, w)


def _add_body(p_ref, o_ref):
    o_ref[...] = p_ref[0] + p_ref[1]


def _add(p):
    _, n, d = p.shape
    return pl.pallas_call(
        _add_body,
        grid=(n // _BN,),
        in_specs=[pl.BlockSpec((2, _BN, d), lambda i: (0, i, 0))],
        out_specs=pl.BlockSpec((_BN, d), lambda i: (i, 0)),
        out_shape=jax.ShapeDtypeStruct((n, d), jnp.float32),
    )(p)


def _addmm_body(p_ref, w_ref, z_ref, t_ref):
    z = p_ref[0] + p_ref[1]
    z_ref[...] = z
    t_ref[...] = jnp.dot(z, w_ref[...], preferred_element_type=jnp.float32)


def _addmm(p, w):
    _, n, din = p.shape
    dout = w.shape[1]
    return pl.pallas_call(
        _addmm_body,
        grid=(n // _BN,),
        in_specs=[pl.BlockSpec((2, _BN, din), lambda i: (0, i, 0)),
                  pl.BlockSpec((din, dout), lambda i: (0, 0))],
        out_specs=[pl.BlockSpec((_BN, din), lambda i: (i, 0)),
                   pl.BlockSpec((_BN, dout), lambda i: (i, 0))],
        out_shape=[jax.ShapeDtypeStruct((n, din), jnp.float32),
                   jax.ShapeDtypeStruct((n, dout), jnp.float32)],
    )(p, w)


_BADJ = 1024


def _adj_body(zi_i, zh_i, zi_j, zh_j, o_ref):
    g1 = _dot_t(zi_i[...], zi_j[...])
    g2 = _dot_t(zh_i[...], zh_j[...])
    o_ref[...] = jax.nn.sigmoid(g1) + jax.nn.sigmoid(g2)


def _adj(zi, zh):
    n, dz = zi.shape
    dh = zh.shape[1]
    return pl.pallas_call(
        _adj_body,
        grid=(n // _BADJ, n // _BADJ),
        in_specs=[pl.BlockSpec((_BADJ, dz), lambda i, j: (i, 0)),
                  pl.BlockSpec((_BADJ, dh), lambda i, j: (i, 0)),
                  pl.BlockSpec((_BADJ, dz), lambda i, j: (j, 0)),
                  pl.BlockSpec((_BADJ, dh), lambda i, j: (j, 0))],
        out_specs=pl.BlockSpec((_BADJ, _BADJ), lambda i, j: (i, j)),
        out_shape=jax.ShapeDtypeStruct((n, n), jnp.float32),
    )(zi, zh, zi, zh)


def _st_body(z_ref, c_ref, o_ref):
    z = z_ref[0]
    c = c_ref[0]
    zz = jnp.sum(z * z, axis=1, keepdims=True)
    cc = jnp.sum(c * c, axis=1).reshape(1, -1)
    d = zz - 2.0 * _dot_t(z, c) + cc
    q = 1.0 / (1.0 + d)
    o_ref[0] = q / jnp.sum(q, axis=1, keepdims=True)


def _student_t4(z4, c2):
    _, n, dz = z4.shape
    ncl = c2.shape[1]
    return pl.pallas_call(
        _st_body,
        grid=(4,),
        in_specs=[pl.BlockSpec((1, n, dz), lambda v: (v, 0, 0)),
                  pl.BlockSpec((1, ncl, dz), lambda v: (v // 2, 0, 0))],
        out_specs=pl.BlockSpec((1, n, ncl), lambda v: (v, 0, 0)),
        out_shape=jax.ShapeDtypeStruct((4, n, ncl), jnp.float32),
    )(z4, c2)


def kernel(x1, adj1_idx, adj1_val, x2, adj2_idx, adj2_val,
           w_e1_1, w_e1_2, w_e1_3, w_d1_1, w_d1_2, w_d1_3,
           w_e2_1, w_e2_2, w_e2_3, w_d2_1, w_d2_2, w_d2_3,
           centers1, centers2):
    ewide = _edges2(adj1_idx, adj1_val, adj2_idx, adj2_val, 128)
    enarrow = _edges2(adj1_idx, adj1_val, adj2_idx, adj2_val, 64)

    ys = jnp.stack([_mm(x1, w_e1_1), _mm(x2, w_e2_1)])
    s = _spmm2(ys, ewide)
    ys = _fuse2(s, jnp.stack([w_e1_2, w_e2_2]))
    s = _spmm2(ys, enarrow)
    ys = _fuse2(s, jnp.stack([w_e1_3, w_e2_3]))
    zi_s, z1_s = _spmm2x(ys, enarrow)
    ys = _fuse2(z1_s, jnp.stack([w_d1_1, w_d2_1]), act=False)
    s = _spmm2(ys, enarrow)
    ys = _fuse2(s, jnp.stack([w_d1_2, w_d2_2]))
    s = _spmm2(ys, ewide)
    t1 = _fuse1(s, 0, w_d1_3)
    t2 = _fuse1(s, 1, w_d2_3)
    zi1, zi2 = zi_s[0], zi_s[1]
    z1, z2 = z1_s[0], z1_s[1]
    z_hat1 = _add(_spmm(t1, adj1_idx, adj1_val))
    a_hat1 = _adj(zi1, z_hat1)
    z_hat2 = _add(_spmm(t2, adj2_idx, adj2_val))
    a_hat2 = _adj(zi2, z_hat2)
    q4 = _student_t4(jnp.stack([z1, zi1, z2, zi2]),
                     jnp.stack([centers1, centers2]))
    Q1 = (q4[0], q4[1])
    Q2 = (q4[2], q4[3])
    return (z_hat1, a_hat1, z_hat2, a_hat2, Q1, Q2, z1, z2, (z1, z2))
